# Initial kernel scaffold; baseline (speedup 1.0000x reference)
import jax
import jax.numpy as jnp
from jax.experimental import pallas as pl


def kernel(x, edge_index, batch, params):
    raise NotImplementedError("write your pallas kernel here")


# speedup vs baseline: 2.9940x; 2.9940x over previous
import functools

import jax
import jax.numpy as jnp
from jax import lax
from jax.experimental import pallas as pl
from jax.experimental.pallas import tpu as pltpu
from jax.experimental.pallas import tpu_sc as plsc

H = 128
G = 64
LG = H // 16


def _ln(x):
    m = jnp.mean(x, axis=-1, keepdims=True)
    v = jnp.mean((x - m) ** 2, axis=-1, keepdims=True)
    return (x - m) * lax.rsqrt(v + 1e-5)


def _dot(a, b):
    return jnp.dot(a, b, preferred_element_type=jnp.float32)


def _input_pool_kernel(x_ref, o_ref, wi_ref, bi_ref, x1_ref, ps_ref, cnt_ref):
    x1 = jnp.tanh(_ln(_dot(x_ref[...], wi_ref[...]) + bi_ref[...]))
    x1_ref[...] = x1
    ot = o_ref[...]

    @pl.when(pl.program_id(0) == 0)
    def _():
        ps_ref[...] = jnp.zeros_like(ps_ref)
        cnt_ref[...] = jnp.zeros_like(cnt_ref)

    ps_ref[...] += lax.dot_general(ot, x1, (((0,), (0,)), ((), ())),
                                   preferred_element_type=jnp.float32)
    cnt_ref[...] += lax.dot_general(
        ot, jnp.ones_like(x1), (((0,), (0,)), ((), ())),
        preferred_element_type=jnp.float32)


def _vp_kernel(ps_ref, cnt_ref, wv1, bv1, wv2, bv2, wv3, bv3, w1c, wnd,
               vp_ref, cv_ref, vd_ref):
    h = ps_ref[...] / cnt_ref[...]
    h = jnp.tanh(_ln(_dot(h, wv1[...]) + bv1[...]))
    h = jnp.tanh(_ln(_dot(h, wv2[...]) + bv2[...]))
    h = jnp.tanh(_ln(_dot(h, wv3[...]) + bv3[...]))
    vp_ref[...] = h
    cv_ref[...] = _dot(h, w1c[...])
    vd_ref[...] = _dot(h, wnd[...])


def _tables_kernel(x_ref, o_ref, cv_ref, vd_ref, w1a, w1b, b1, wna, wnb, wnc,
                   bn, a_ref, b_ref, p_ref, q_ref, r_ref):
    x = x_ref[...]
    o = o_ref[...]
    a_ref[...] = _dot(x, w1a[...]) + _dot(o, cv_ref[...]) + b1[...]
    b_ref[...] = _dot(x, w1b[...])
    p_ref[...] = _dot(x, wna[...])
    q_ref[...] = _dot(x, wnb[...])
    r_ref[...] = _dot(x, wnc[...]) + _dot(o, vd_ref[...]) + bn[...]


def _edge_tail_kernel(h1_ref, w2, b2, w3, b3, w4r, b4, e_ref):
    u = jnp.tanh(_ln(h1_ref[...]))
    u = jnp.tanh(_ln(_dot(u, w2[...]) + b2[...]))
    u = jnp.tanh(_ln(_dot(u, w3[...]) + b3[...]))
    logit = jnp.sum(u * w4r[...], axis=-1, keepdims=True) + b4[...]
    e_ref[...] = 1.0 / (1.0 + jnp.exp(-logit))


def _node_kernel(agga_ref, aggb_ref, rt_ref, xin_ref, o_ref, wn2, bn2, wn3,
                 bn3, wn4, bn4, xo_ref, ps_ref):
    h = jnp.tanh(_ln(agga_ref[...] + aggb_ref[...] + rt_ref[...]))
    h = jnp.tanh(_ln(_dot(h, wn2[...]) + bn2[...]))
    h = jnp.tanh(_ln(_dot(h, wn3[...]) + bn3[...]))
    h = jnp.tanh(_ln(_dot(h, wn4[...]) + bn4[...]))

    @pl.when(pl.program_id(0) == 0)
    def _():
        ps_ref[...] = jnp.zeros_like(ps_ref)

    ps_ref[...] += lax.dot_general(o_ref[...], h, (((0,), (0,)), ((), ())),
                                   preferred_element_type=jnp.float32)
    xo_ref[...] = h + xin_ref[...]


def _ip_kernel(v0, v1, v2, v3, wp1, bp1, wp2, bp2, wp3, bp3, out_ref):
    h = jnp.concatenate([v0[...], v1[...], v2[...], v3[...]], axis=1)
    h = jnp.tanh(_ln(_dot(h, wp1[...]) + bp1[...]))
    h = jnp.tanh(_ln(_dot(h, wp2[...]) + bp2[...]))
    h = jnp.tanh(_ln(_dot(h, wp3[...]) + bp3[...]))
    out_ref[...] = h


def _full(shape):
    return pl.BlockSpec(shape, lambda i: (0,) * len(shape))


def _rows(bs, width):
    return pl.BlockSpec((bs, width), lambda i: (i, 0))


def _sc_gather(A, B, start, end):
    E = start.shape[0]
    info = plsc.get_sparse_core_info()
    NW = info.num_cores * info.num_subcores
    Ew = E // NW
    C = 128
    nch = (Ew + C - 1) // C
    mesh = plsc.VectorSubcoreMesh(core_axis_name="c", subcore_axis_name="s")

    @functools.partial(
        pl.kernel,
        out_type=jax.ShapeDtypeStruct((E, H), jnp.float32),
        mesh=mesh,
        scratch_types=[
            pltpu.VMEM((C,), jnp.int32),
            pltpu.VMEM((C,), jnp.int32),
            pltpu.VMEM((C, H), jnp.float32),
            pltpu.VMEM((C, H), jnp.float32),
            pltpu.SemaphoreType.DMA,
            pltpu.SemaphoreType.DMA,
        ],
    )
    def k(a_hbm, b_hbm, s_hbm, e_hbm, out_hbm, idx_s, idx_e, buf_a, buf_b,
          sem_a, sem_b):
        wid = lax.axis_index("s") * info.num_cores + lax.axis_index("c")
        base = wid * Ew

        def chunk(kk, carry):
            off = base + jnp.minimum(kk * C, Ew - C)
            pltpu.sync_copy(s_hbm.at[pl.ds(off, C)], idx_s)
            pltpu.sync_copy(e_hbm.at[pl.ds(off, C)], idx_e)
            cpa = pltpu.async_copy(a_hbm.at[idx_s], buf_a, sem_a)
            cpb = pltpu.async_copy(b_hbm.at[idx_e], buf_b, sem_b)
            cpa.wait()
            cpb.wait()

            def body(j, c2):
                for l in range(LG):
                    sl = pl.ds(l * 16, 16)
                    buf_a[j, sl] = buf_a[j, sl] + buf_b[j, sl]
                return c2

            lax.fori_loop(0, C, body, 0)
            pltpu.sync_copy(buf_a, out_hbm.at[pl.ds(off, C)])
            return carry

        lax.fori_loop(0, nch, chunk, 0)

    return k(A, B, start, end)


def _sc_scatter(P, Q, e, start, end):
    E = start.shape[0]
    N = P.shape[0]
    info = plsc.get_sparse_core_info()
    NC, NS = info.num_cores, info.num_subcores
    NW = NC * NS
    Ew = E // NW
    C = 80
    nch = Ew // C
    ZR = 128
    rows_per_tile = (((N + NS - 1) // NS + ZR - 1) // ZR) * ZR
    NP = rows_per_tile * NS
    nz = rows_per_tile // ZR
    mesh = plsc.VectorSubcoreMesh(core_axis_name="c", subcore_axis_name="s")

    @functools.partial(
        pl.kernel,
        out_type=jax.ShapeDtypeStruct((NC, NP, H), jnp.float32),
        mesh=mesh,
        scratch_types=[
            pltpu.VMEM((C,), jnp.int32),
            pltpu.VMEM((C,), jnp.int32),
            pltpu.VMEM((C,), jnp.float32),
            pltpu.VMEM((C, H), jnp.float32),
            pltpu.VMEM((C, H), jnp.float32),
            pltpu.VMEM((ZR, H), jnp.float32),
            pltpu.VMEM_SHARED((NP, H), jnp.float32),
            pltpu.SemaphoreType.DMA,
            pltpu.SemaphoreType.DMA,
        ],
    )
    def k(p_hbm, q_hbm, ev_hbm, s_hbm, e_hbm, out_hbm, idx_s, idx_e, ebuf,
          buf_p, buf_q, zbuf, acc, sem_p, sem_q):
        cid = lax.axis_index("c")
        sid = lax.axis_index("s")
        wid = sid * NC + cid
        base = wid * Ew
        r0 = sid * rows_per_tile

        def zb(j, c2):
            for l in range(LG):
                zbuf[j, pl.ds(l * 16, 16)] = jnp.zeros((16,), jnp.float32)
            return c2

        lax.fori_loop(0, ZR, zb, 0)
        for i in range(nz):
            pltpu.sync_copy(zbuf, acc.at[pl.ds(r0 + i * ZR, ZR)])
        plsc.subcore_barrier()

        def chunk(kk, carry):
            off = base + kk * C
            pltpu.sync_copy(s_hbm.at[pl.ds(off, C)], idx_s)
            pltpu.sync_copy(e_hbm.at[pl.ds(off, C)], idx_e)
            pltpu.sync_copy(ev_hbm.at[pl.ds(off, C)], ebuf)
            cpp = pltpu.async_copy(p_hbm.at[idx_s], buf_p, sem_p)
            cpq = pltpu.async_copy(q_hbm.at[idx_e], buf_q, sem_q)
            cpp.wait()
            cpq.wait()

            def body(g, c2):
                ev = ebuf[pl.ds(g * 16, 16)]
                for l in range(16):
                    bv = jnp.full((16,), ev[l], jnp.float32)
                    j = g * 16 + l
                    for lg in range(LG):
                        sl = pl.ds(lg * 16, 16)
                        buf_p[j, sl] = buf_p[j, sl] * bv
                        buf_q[j, sl] = buf_q[j, sl] * bv
                return c2

            lax.fori_loop(0, C // 16, body, 0)
            pltpu.sync_copy(buf_p, acc.at[idx_e], add=True)
            pltpu.sync_copy(buf_q, acc.at[idx_s], add=True)
            return carry

        lax.fori_loop(0, nch, chunk, 0)
        plsc.subcore_barrier()
        for i in range(nz):
            rr = r0 + i * ZR
            pltpu.sync_copy(acc.at[pl.ds(rr, ZR)], out_hbm.at[cid, pl.ds(rr, ZR)])

    return k(P, Q, e, start, end)[:, :N, :]


def kernel(x, edge_index, batch, params):
    N = x.shape[0]
    E = edge_index.shape[1]
    start = edge_index[0]
    end = edge_index[1]
    O = (batch[:, None] == jnp.arange(G, dtype=batch.dtype)[None, :]).astype(
        jnp.float32)

    RN = 1000
    NB = N // RN
    RE = 512
    EB = E // RE

    (wi, bi), = params["input"]
    (w1, b1), (w2, b2), (w3, b3), (w4, b4) = params["edge"]
    (wn1, bn1), (wn2, bn2), (wn3, bn3), (wn4, bn4) = params["node"]
    (wv1, bv1), (wv2, bv2), (wv3, bv3) = params["vp"]
    (wp1, bp1), (wp2, bp2), (wp3, bp3) = params["ip"]

    w1a, w1b, w1c = w1[:H], w1[H:2 * H], w1[2 * H:]
    wna, wnb, wnc, wnd = wn1[:H], wn1[H:2 * H], wn1[2 * H:3 * H], wn1[3 * H:]
    row = lambda v: v.reshape(1, -1)
    bi_, b1_, bn1_ = row(bi), row(b1), row(bn1)
    b2_, b3_, b4_ = row(b2), row(b3), row(b4)
    bn2_, bn3_, bn4_ = row(bn2), row(bn3), row(bn4)
    bv1_, bv2_, bv3_ = row(bv1), row(bv2), row(bv3)
    bp1_, bp2_, bp3_ = row(bp1), row(bp2), row(bp3)
    w4r = w4.reshape(1, H)

    wspec = _full((H, H))
    bspec = _full((1, H))
    gspec = _full((G, H))

    x1, ps, cnt = pl.pallas_call(
        _input_pool_kernel,
        grid=(NB,),
        in_specs=[_rows(RN, H), _rows(RN, G), wspec, bspec],
        out_specs=[_rows(RN, H), gspec, gspec],
        out_shape=[
            jax.ShapeDtypeStruct((N, H), jnp.float32),
            jax.ShapeDtypeStruct((G, H), jnp.float32),
            jax.ShapeDtypeStruct((G, H), jnp.float32),
        ],
    )(x, O, wi, bi_)

    def vp_net(psum, cntf):
        return pl.pallas_call(
            _vp_kernel,
            grid=(1,),
            in_specs=[gspec] * 2 + [wspec, bspec] * 3 + [wspec, wspec],
            out_specs=[gspec] * 3,
            out_shape=[jax.ShapeDtypeStruct((G, H), jnp.float32)] * 3,
        )(psum, cntf, wv1, bv1_, wv2, bv2_, wv3, bv3_, w1c, wnd)

    vp, cv, vd = vp_net(ps, cnt)
    vp_all = [vp]

    xc = x1
    e_col = None
    for _ in range(3):
        A, B, P, Q, Rt = pl.pallas_call(
            _tables_kernel,
            grid=(NB,),
            in_specs=[_rows(RN, H), _rows(RN, G), gspec, gspec,
                      wspec, wspec, bspec, wspec, wspec, wspec, bspec],
            out_specs=[_rows(RN, H)] * 5,
            out_shape=[jax.ShapeDtypeStruct((N, H), jnp.float32)] * 5,
        )(xc, O, cv, vd, w1a, w1b, b1_, wna, wnb, wnc, bn1_)

        h1 = _sc_gather(A, B, start, end)

        e_col = pl.pallas_call(
            _edge_tail_kernel,
            grid=(EB,),
            in_specs=[_rows(RE, H), wspec, bspec, wspec, bspec, bspec,
                      _full((1, 1))],
            out_specs=_rows(RE, 1),
            out_shape=jax.ShapeDtypeStruct((E, 1), jnp.float32),
        )(h1, w2, b2_, w3, b3_, w4r, b4_)

        partials = _sc_scatter(P, Q, e_col.reshape(E), start, end)

        xc, ps = pl.pallas_call(
            _node_kernel,
            grid=(NB,),
            in_specs=[_rows(RN, H)] * 4 + [_rows(RN, G)] +
                     [wspec, bspec] * 3,
            out_specs=[_rows(RN, H), gspec],
            out_shape=[
                jax.ShapeDtypeStruct((N, H), jnp.float32),
                jax.ShapeDtypeStruct((G, H), jnp.float32),
            ],
        )(partials[0], partials[1], Rt, xc, O, wn2, bn2_, wn3, bn3_, wn4,
          bn4_)

        vp, cv, vd = vp_net(ps, cnt)
        vp_all.append(vp)

    ip = pl.pallas_call(
        _ip_kernel,
        grid=(1,),
        in_specs=[gspec] * 4 + [_full((4 * H, H)), bspec] +
                 [wspec, bspec] * 2,
        out_specs=gspec,
        out_shape=jax.ShapeDtypeStruct((G, H), jnp.float32),
    )(vp_all[0], vp_all[1], vp_all[2], vp_all[3], wp1, bp1_, wp2, bp2_,
      wp3, bp3_)

    return (e_col.reshape(E), xc, ip)


# speedup vs baseline: 4.0582x; 1.3554x over previous
import functools

import jax
import jax.numpy as jnp
from jax import lax
from jax.experimental import pallas as pl
from jax.experimental.pallas import tpu as pltpu
from jax.experimental.pallas import tpu_sc as plsc

H = 128
G = 64
LG = H // 16


def _ln(x):
    m = jnp.mean(x, axis=-1, keepdims=True)
    v = jnp.mean((x - m) ** 2, axis=-1, keepdims=True)
    return (x - m) * lax.rsqrt(v + 1e-5)


def _dot(a, b):
    return jnp.dot(a, b, preferred_element_type=jnp.float32)


def _input_pool_kernel(x_ref, o_ref, wi_ref, bi_ref, x1_ref, ps_ref, cnt_ref):
    x1 = jnp.tanh(_ln(_dot(x_ref[...], wi_ref[...]) + bi_ref[...]))
    x1_ref[...] = x1
    ot = o_ref[...]

    @pl.when(pl.program_id(0) == 0)
    def _():
        ps_ref[...] = jnp.zeros_like(ps_ref)
        cnt_ref[...] = jnp.zeros_like(cnt_ref)

    ps_ref[...] += lax.dot_general(ot, x1, (((0,), (0,)), ((), ())),
                                   preferred_element_type=jnp.float32)
    cnt_ref[...] += lax.dot_general(
        ot, jnp.ones_like(x1), (((0,), (0,)), ((), ())),
        preferred_element_type=jnp.float32)


def _vp_kernel(ps_ref, cnt_ref, wv1, bv1, wv2, bv2, wv3, bv3, w1c, wnd,
               vp_ref, cv_ref, vd_ref):
    h = ps_ref[...] / cnt_ref[...]
    h = jnp.tanh(_ln(_dot(h, wv1[...]) + bv1[...]))
    h = jnp.tanh(_ln(_dot(h, wv2[...]) + bv2[...]))
    h = jnp.tanh(_ln(_dot(h, wv3[...]) + bv3[...]))
    vp_ref[...] = h
    cv_ref[...] = _dot(h, w1c[...])
    vd_ref[...] = _dot(h, wnd[...])


def _tables_kernel(x_ref, o_ref, cv_ref, vd_ref, w1a, w1b, b1, wna, wnb, wnc,
                   bn, a_ref, b_ref, p_ref, q_ref, r_ref):
    x = x_ref[...]
    o = o_ref[...]
    a_ref[...] = _dot(x, w1a[...]) + _dot(o, cv_ref[...]) + b1[...]
    b_ref[...] = _dot(x, w1b[...])
    p_ref[...] = _dot(x, wna[...])
    q_ref[...] = _dot(x, wnb[...])
    r_ref[...] = _dot(x, wnc[...]) + _dot(o, vd_ref[...]) + bn[...]


def _edge_tail_kernel(h1_ref, w2, b2, w3, b3, w4r, b4, e_ref):
    u = jnp.tanh(_ln(h1_ref[...]))
    u = jnp.tanh(_ln(_dot(u, w2[...]) + b2[...]))
    u = jnp.tanh(_ln(_dot(u, w3[...]) + b3[...]))
    logit = jnp.sum(u * w4r[...], axis=-1, keepdims=True) + b4[...]
    e_ref[...] = 1.0 / (1.0 + jnp.exp(-logit))


def _node_kernel(agga_ref, aggb_ref, rt_ref, xin_ref, o_ref, wn2, bn2, wn3,
                 bn3, wn4, bn4, xo_ref, ps_ref):
    h = jnp.tanh(_ln(agga_ref[...] + aggb_ref[...] + rt_ref[...]))
    h = jnp.tanh(_ln(_dot(h, wn2[...]) + bn2[...]))
    h = jnp.tanh(_ln(_dot(h, wn3[...]) + bn3[...]))
    h = jnp.tanh(_ln(_dot(h, wn4[...]) + bn4[...]))

    @pl.when(pl.program_id(0) == 0)
    def _():
        ps_ref[...] = jnp.zeros_like(ps_ref)

    ps_ref[...] += lax.dot_general(o_ref[...], h, (((0,), (0,)), ((), ())),
                                   preferred_element_type=jnp.float32)
    xo_ref[...] = h + xin_ref[...]


def _ip_kernel(v0, v1, v2, v3, wp1, bp1, wp2, bp2, wp3, bp3, out_ref):
    h = jnp.concatenate([v0[...], v1[...], v2[...], v3[...]], axis=1)
    h = jnp.tanh(_ln(_dot(h, wp1[...]) + bp1[...]))
    h = jnp.tanh(_ln(_dot(h, wp2[...]) + bp2[...]))
    h = jnp.tanh(_ln(_dot(h, wp3[...]) + bp3[...]))
    out_ref[...] = h


def _full(shape):
    return pl.BlockSpec(shape, lambda i: (0,) * len(shape))


def _rows(bs, width):
    return pl.BlockSpec((bs, width), lambda i: (i, 0))


def _sc_gather(A, B, start, end):
    E = start.shape[0]
    info = plsc.get_sparse_core_info()
    NW = info.num_cores * info.num_subcores
    Ew = E // NW
    C = 128
    nch = 2 * ((Ew + 2 * C - 1) // (2 * C))
    mesh = plsc.VectorSubcoreMesh(core_axis_name="c", subcore_axis_name="s")

    @functools.partial(
        pl.kernel,
        out_type=jax.ShapeDtypeStruct((E, H), jnp.float32),
        mesh=mesh,
        scratch_types=[
            [pltpu.VMEM((C,), jnp.int32)] * 2,
            [pltpu.VMEM((C,), jnp.int32)] * 2,
            [pltpu.VMEM((C, H), jnp.float32)] * 2,
            [pltpu.VMEM((C, H), jnp.float32)] * 2,
            [pltpu.VMEM((C, H), jnp.float32)] * 2,
            [pltpu.SemaphoreType.DMA] * 2,
            [pltpu.SemaphoreType.DMA] * 2,
            [pltpu.SemaphoreType.DMA] * 2,
        ],
    )
    def k(a_hbm, b_hbm, s_hbm, e_hbm, out_hbm, idx_s, idx_e, buf_a, buf_b,
          buf_o, isem, gsem, osem):
        wid = lax.axis_index("s") * info.num_cores + lax.axis_index("c")
        base = wid * Ew
        off_of = lambda c: base + jnp.minimum(c * C, Ew - C)

        def issue_idx(c, b):
            pltpu.async_copy(s_hbm.at[pl.ds(off_of(c), C)], idx_s[b], isem[b])
            pltpu.async_copy(e_hbm.at[pl.ds(off_of(c), C)], idx_e[b], isem[b])

        def wait_idx(b):
            pltpu.make_async_copy(s_hbm.at[pl.ds(0, C)], idx_s[b], isem[b]).wait()
            pltpu.make_async_copy(e_hbm.at[pl.ds(0, C)], idx_e[b], isem[b]).wait()

        def issue_gather(b):
            pltpu.async_copy(a_hbm.at[idx_s[b]], buf_a[b], gsem[b])
            pltpu.async_copy(b_hbm.at[idx_e[b]], buf_b[b], gsem[b])

        def wait_gather(b):
            pltpu.make_async_copy(a_hbm.at[idx_s[b]], buf_a[b], gsem[b]).wait()
            pltpu.make_async_copy(b_hbm.at[idx_e[b]], buf_b[b], gsem[b]).wait()

        def wait_store(b):
            pltpu.make_async_copy(buf_o[b], out_hbm.at[pl.ds(0, C)], osem[b]).wait()

        issue_idx(0, 0)
        issue_idx(1, 1)
        wait_idx(0)
        issue_gather(0)

        def step(kk, carry):
            for b in (0, 1):
                c = 2 * kk + b
                b1 = 1 - b
                wait_gather(b)

                @pl.when(c + 1 < nch)
                def _():
                    wait_idx(b1)
                    issue_gather(b1)

                @pl.when(c + 2 < nch)
                def _():
                    issue_idx(c + 2, b)

                @pl.when(c >= 2)
                def _():
                    wait_store(b)

                def body(j, c2):
                    for l in range(LG):
                        sl = pl.ds(l * 16, 16)
                        buf_o[b][j, sl] = buf_a[b][j, sl] + buf_b[b][j, sl]
                    return c2

                lax.fori_loop(0, C, body, 0)
                pltpu.async_copy(buf_o[b], out_hbm.at[pl.ds(off_of(c), C)],
                                 osem[b])
            return carry

        lax.fori_loop(0, nch // 2, step, 0)
        wait_store(0)
        wait_store(1)

    return k(A, B, start, end)


def _sc_scatter(P, Q, e, start, end):
    E = start.shape[0]
    N = P.shape[0]
    info = plsc.get_sparse_core_info()
    NC, NS = info.num_cores, info.num_subcores
    NW = NC * NS
    Ew = E // NW
    C = 80
    nch = Ew // C
    ZR = 32
    rows_per_tile = (((N + NS - 1) // NS + ZR - 1) // ZR) * ZR
    NP = rows_per_tile * NS
    nz = rows_per_tile // ZR
    mesh = plsc.VectorSubcoreMesh(core_axis_name="c", subcore_axis_name="s")

    nch2 = 2 * ((nch + 1) // 2)

    @functools.partial(
        pl.kernel,
        out_type=jax.ShapeDtypeStruct((NC, NP, H), jnp.float32),
        mesh=mesh,
        scratch_types=[
            [pltpu.VMEM((C,), jnp.int32)] * 2,
            [pltpu.VMEM((C,), jnp.int32)] * 2,
            [pltpu.VMEM((C,), jnp.float32)] * 2,
            [pltpu.VMEM((C, H), jnp.float32)] * 2,
            [pltpu.VMEM((C, H), jnp.float32)] * 2,
            pltpu.VMEM((ZR, H), jnp.float32),
            pltpu.VMEM_SHARED((NP, H), jnp.float32),
            [pltpu.SemaphoreType.DMA] * 2,
            [pltpu.SemaphoreType.DMA] * 2,
        ],
    )
    def k(p_hbm, q_hbm, ev_hbm, s_hbm, e_hbm, out_hbm, idx_s, idx_e, ebuf,
          buf_p, buf_q, zbuf, acc, isem, gsem):
        cid = lax.axis_index("c")
        sid = lax.axis_index("s")
        wid = sid * NC + cid
        base = wid * Ew
        r0 = sid * rows_per_tile

        def issue_idx(c, b):
            off = base + c * C
            pltpu.async_copy(s_hbm.at[pl.ds(off, C)], idx_s[b], isem[b])
            pltpu.async_copy(e_hbm.at[pl.ds(off, C)], idx_e[b], isem[b])
            pltpu.async_copy(ev_hbm.at[pl.ds(off, C)], ebuf[b], isem[b])

        def wait_idx(b):
            pltpu.make_async_copy(s_hbm.at[pl.ds(0, C)], idx_s[b], isem[b]).wait()
            pltpu.make_async_copy(e_hbm.at[pl.ds(0, C)], idx_e[b], isem[b]).wait()
            pltpu.make_async_copy(ev_hbm.at[pl.ds(0, C)], ebuf[b], isem[b]).wait()

        def issue_gather(b):
            pltpu.async_copy(p_hbm.at[idx_s[b]], buf_p[b], gsem[b])
            pltpu.async_copy(q_hbm.at[idx_e[b]], buf_q[b], gsem[b])

        def wait_gather(b):
            pltpu.make_async_copy(p_hbm.at[idx_s[b]], buf_p[b], gsem[b]).wait()
            pltpu.make_async_copy(q_hbm.at[idx_e[b]], buf_q[b], gsem[b]).wait()

        def zb(j, c2):
            for l in range(LG):
                zbuf[j, pl.ds(l * 16, 16)] = jnp.zeros((16,), jnp.float32)
            return c2

        lax.fori_loop(0, ZR, zb, 0)
        for i in range(nz):
            pltpu.sync_copy(zbuf, acc.at[pl.ds(r0 + i * ZR, ZR)])
        plsc.subcore_barrier()

        issue_idx(0, 0)
        issue_idx(1, 1)
        wait_idx(0)
        issue_gather(0)

        def step(kk, carry):
            for b in (0, 1):
                c = 2 * kk + b
                b1 = 1 - b

                @pl.when(c < nch)
                def _():
                    wait_gather(b)

                    @pl.when(c + 1 < nch)
                    def _():
                        wait_idx(b1)
                        issue_gather(b1)

                    def body(g, c2):
                        ev = ebuf[b][pl.ds(g * 16, 16)]
                        for l in range(16):
                            bv = jnp.full((16,), ev[l], jnp.float32)
                            j = g * 16 + l
                            for lg in range(LG):
                                sl = pl.ds(lg * 16, 16)
                                buf_p[b][j, sl] = buf_p[b][j, sl] * bv
                                buf_q[b][j, sl] = buf_q[b][j, sl] * bv
                        return c2

                    lax.fori_loop(0, C // 16, body, 0)
                    pltpu.sync_copy(buf_p[b], acc.at[idx_e[b]], add=True)
                    pltpu.sync_copy(buf_q[b], acc.at[idx_s[b]], add=True)

                    @pl.when(c + 2 < nch)
                    def _():
                        issue_idx(c + 2, b)
            return carry

        lax.fori_loop(0, nch2 // 2, step, 0)
        plsc.subcore_barrier()
        for i in range(nz):
            rr = r0 + i * ZR
            pltpu.sync_copy(acc.at[pl.ds(rr, ZR)], out_hbm.at[cid, pl.ds(rr, ZR)])

    return k(P, Q, e, start, end)[:, :N, :]


def kernel(x, edge_index, batch, params):
    N = x.shape[0]
    E = edge_index.shape[1]
    start = edge_index[0]
    end = edge_index[1]
    O = (batch[:, None] == jnp.arange(G, dtype=batch.dtype)[None, :]).astype(
        jnp.float32)

    RN = 1000
    NB = N // RN
    RE = 512
    EB = E // RE

    (wi, bi), = params["input"]
    (w1, b1), (w2, b2), (w3, b3), (w4, b4) = params["edge"]
    (wn1, bn1), (wn2, bn2), (wn3, bn3), (wn4, bn4) = params["node"]
    (wv1, bv1), (wv2, bv2), (wv3, bv3) = params["vp"]
    (wp1, bp1), (wp2, bp2), (wp3, bp3) = params["ip"]

    w1a, w1b, w1c = w1[:H], w1[H:2 * H], w1[2 * H:]
    wna, wnb, wnc, wnd = wn1[:H], wn1[H:2 * H], wn1[2 * H:3 * H], wn1[3 * H:]
    row = lambda v: v.reshape(1, -1)
    bi_, b1_, bn1_ = row(bi), row(b1), row(bn1)
    b2_, b3_, b4_ = row(b2), row(b3), row(b4)
    bn2_, bn3_, bn4_ = row(bn2), row(bn3), row(bn4)
    bv1_, bv2_, bv3_ = row(bv1), row(bv2), row(bv3)
    bp1_, bp2_, bp3_ = row(bp1), row(bp2), row(bp3)
    w4r = w4.reshape(1, H)

    wspec = _full((H, H))
    bspec = _full((1, H))
    gspec = _full((G, H))

    x1, ps, cnt = pl.pallas_call(
        _input_pool_kernel,
        grid=(NB,),
        in_specs=[_rows(RN, H), _rows(RN, G), wspec, bspec],
        out_specs=[_rows(RN, H), gspec, gspec],
        out_shape=[
            jax.ShapeDtypeStruct((N, H), jnp.float32),
            jax.ShapeDtypeStruct((G, H), jnp.float32),
            jax.ShapeDtypeStruct((G, H), jnp.float32),
        ],
    )(x, O, wi, bi_)

    def vp_net(psum, cntf):
        return pl.pallas_call(
            _vp_kernel,
            grid=(1,),
            in_specs=[gspec] * 2 + [wspec, bspec] * 3 + [wspec, wspec],
            out_specs=[gspec] * 3,
            out_shape=[jax.ShapeDtypeStruct((G, H), jnp.float32)] * 3,
        )(psum, cntf, wv1, bv1_, wv2, bv2_, wv3, bv3_, w1c, wnd)

    vp, cv, vd = vp_net(ps, cnt)
    vp_all = [vp]

    xc = x1
    e_col = None
    for _ in range(3):
        A, B, P, Q, Rt = pl.pallas_call(
            _tables_kernel,
            grid=(NB,),
            in_specs=[_rows(RN, H), _rows(RN, G), gspec, gspec,
                      wspec, wspec, bspec, wspec, wspec, wspec, bspec],
            out_specs=[_rows(RN, H)] * 5,
            out_shape=[jax.ShapeDtypeStruct((N, H), jnp.float32)] * 5,
        )(xc, O, cv, vd, w1a, w1b, b1_, wna, wnb, wnc, bn1_)

        h1 = _sc_gather(A, B, start, end)

        e_col = pl.pallas_call(
            _edge_tail_kernel,
            grid=(EB,),
            in_specs=[_rows(RE, H), wspec, bspec, wspec, bspec, bspec,
                      _full((1, 1))],
            out_specs=_rows(RE, 1),
            out_shape=jax.ShapeDtypeStruct((E, 1), jnp.float32),
        )(h1, w2, b2_, w3, b3_, w4r, b4_)

        partials = _sc_scatter(P, Q, e_col.reshape(E), start, end)

        xc, ps = pl.pallas_call(
            _node_kernel,
            grid=(NB,),
            in_specs=[_rows(RN, H)] * 4 + [_rows(RN, G)] +
                     [wspec, bspec] * 3,
            out_specs=[_rows(RN, H), gspec],
            out_shape=[
                jax.ShapeDtypeStruct((N, H), jnp.float32),
                jax.ShapeDtypeStruct((G, H), jnp.float32),
            ],
        )(partials[0], partials[1], Rt, xc, O, wn2, bn2_, wn3, bn3_, wn4,
          bn4_)

        vp, cv, vd = vp_net(ps, cnt)
        vp_all.append(vp)

    ip = pl.pallas_call(
        _ip_kernel,
        grid=(1,),
        in_specs=[gspec] * 4 + [_full((4 * H, H)), bspec] +
                 [wspec, bspec] * 2,
        out_specs=gspec,
        out_shape=jax.ShapeDtypeStruct((G, H), jnp.float32),
    )(vp_all[0], vp_all[1], vp_all[2], vp_all[3], wp1, bp1_, wp2, bp2_,
      wp3, bp3_)

    return (e_col.reshape(E), xc, ip)


# speedup vs baseline: 5.0329x; 1.2402x over previous
import functools

import jax
import jax.numpy as jnp
from jax import lax
from jax.experimental import pallas as pl
from jax.experimental.pallas import tpu as pltpu
from jax.experimental.pallas import tpu_sc as plsc

H = 128
G = 64
LG = H // 16


def _ln(x):
    m = jnp.mean(x, axis=-1, keepdims=True)
    v = jnp.mean((x - m) ** 2, axis=-1, keepdims=True)
    return (x - m) * lax.rsqrt(v + 1e-5)


def _dot(a, b):
    return jnp.dot(a, b, preferred_element_type=jnp.float32)


def _input_pool_kernel(x_ref, o_ref, wi_ref, bi_ref, x1_ref, ps_ref, cnt_ref):
    x1 = jnp.tanh(_ln(_dot(x_ref[...], wi_ref[...]) + bi_ref[...]))
    x1_ref[...] = x1
    ot = o_ref[...]

    @pl.when(pl.program_id(0) == 0)
    def _():
        ps_ref[...] = jnp.zeros_like(ps_ref)
        cnt_ref[...] = jnp.zeros_like(cnt_ref)

    ps_ref[...] += lax.dot_general(ot, x1, (((0,), (0,)), ((), ())),
                                   preferred_element_type=jnp.float32)
    cnt_ref[...] += lax.dot_general(
        ot, jnp.ones_like(x1), (((0,), (0,)), ((), ())),
        preferred_element_type=jnp.float32)


def _vp_mlp(ps, cnt, wv1, bv1, wv2, bv2, wv3, bv3):
    h = ps / cnt
    h = jnp.tanh(_ln(_dot(h, wv1[...]) + bv1[...]))
    h = jnp.tanh(_ln(_dot(h, wv2[...]) + bv2[...]))
    return jnp.tanh(_ln(_dot(h, wv3[...]) + bv3[...]))


def _tables_kernel(x_ref, o_ref, ps_ref, cnt_ref, wv1, bv1, wv2, bv2, wv3,
                   bv3, w1c, wnd, w1a, w1b, b1, wna, wnb, wnc, bn,
                   vp_ref, cv_ref, vd_ref, a_ref, b_ref, p_ref, q_ref, r_ref):
    @pl.when(pl.program_id(0) == 0)
    def _():
        vp = _vp_mlp(ps_ref[...], cnt_ref[...], wv1, bv1, wv2, bv2, wv3, bv3)
        vp_ref[...] = vp
        cv_ref[...] = _dot(vp, w1c[...])
        vd_ref[...] = _dot(vp, wnd[...])

    x = x_ref[...]
    o = o_ref[...]
    a_ref[...] = _dot(x, w1a[...]) + _dot(o, cv_ref[...]) + b1[...]
    b_ref[...] = _dot(x, w1b[...])
    p_ref[...] = _dot(x, wna[...])
    q_ref[...] = _dot(x, wnb[...])
    r_ref[...] = _dot(x, wnc[...]) + _dot(o, vd_ref[...]) + bn[...]


def _edge_tail_kernel(h1_ref, w2, b2, w3, b3, w4r, b4, e_ref):
    u = jnp.tanh(_ln(h1_ref[...]))
    u = _dot(u.astype(jnp.bfloat16), w2[...]) + b2[...]
    u = jnp.tanh(_ln(u))
    u = _dot(u.astype(jnp.bfloat16), w3[...]) + b3[...]
    u = jnp.tanh(_ln(u))
    logit = jnp.sum(u * w4r[...], axis=-1, keepdims=True) + b4[...]
    e_ref[...] = 1.0 / (1.0 + jnp.exp(-logit))


def _node_kernel(agga_ref, aggb_ref, rt_ref, xin_ref, o_ref, wn2, bn2, wn3,
                 bn3, wn4, bn4, xo_ref, ps_ref):
    h = jnp.tanh(_ln(agga_ref[...] + aggb_ref[...] + rt_ref[...]))
    h = jnp.tanh(_ln(_dot(h, wn2[...]) + bn2[...]))
    h = jnp.tanh(_ln(_dot(h, wn3[...]) + bn3[...]))
    h = jnp.tanh(_ln(_dot(h, wn4[...]) + bn4[...]))

    @pl.when(pl.program_id(0) == 0)
    def _():
        ps_ref[...] = jnp.zeros_like(ps_ref)

    ps_ref[...] += lax.dot_general(o_ref[...], h, (((0,), (0,)), ((), ())),
                                   preferred_element_type=jnp.float32)
    xo_ref[...] = h + xin_ref[...]


def _ip_kernel(v0, v1, v2, ps_ref, cnt_ref, wv1, bv1, wv2, bv2, wv3, bv3,
               wp1, bp1, wp2, bp2, wp3, bp3, out_ref):
    v3 = _vp_mlp(ps_ref[...], cnt_ref[...], wv1, bv1, wv2, bv2, wv3, bv3)
    h = jnp.concatenate([v0[...], v1[...], v2[...], v3], axis=1)
    h = jnp.tanh(_ln(_dot(h, wp1[...]) + bp1[...]))
    h = jnp.tanh(_ln(_dot(h, wp2[...]) + bp2[...]))
    h = jnp.tanh(_ln(_dot(h, wp3[...]) + bp3[...]))
    out_ref[...] = h


def _full(shape):
    return pl.BlockSpec(shape, lambda i: (0,) * len(shape))


def _rows(bs, width):
    return pl.BlockSpec((bs, width), lambda i: (i, 0))


def _sc_gather(A, B, start, end):
    E = start.shape[0]
    info = plsc.get_sparse_core_info()
    NW = info.num_cores * info.num_subcores
    Ew = E // NW
    C = 128
    nch = 2 * ((Ew + 2 * C - 1) // (2 * C))
    mesh = plsc.VectorSubcoreMesh(core_axis_name="c", subcore_axis_name="s")

    @functools.partial(
        pl.kernel,
        out_type=jax.ShapeDtypeStruct((E, H), jnp.float32),
        mesh=mesh,
        scratch_types=[
            [pltpu.VMEM((C,), jnp.int32)] * 2,
            [pltpu.VMEM((C,), jnp.int32)] * 2,
            [pltpu.VMEM((C, H), jnp.float32)] * 2,
            [pltpu.VMEM((C, H), jnp.float32)] * 2,
            [pltpu.VMEM((C, H), jnp.float32)] * 2,
            [pltpu.SemaphoreType.DMA] * 2,
            [pltpu.SemaphoreType.DMA] * 2,
            [pltpu.SemaphoreType.DMA] * 2,
        ],
    )
    def k(a_hbm, b_hbm, s_hbm, e_hbm, out_hbm, idx_s, idx_e, buf_a, buf_b,
          buf_o, isem, gsem, osem):
        wid = lax.axis_index("s") * info.num_cores + lax.axis_index("c")
        base = wid * Ew
        off_of = lambda c: base + jnp.minimum(c * C, Ew - C)

        def issue_idx(c, b):
            pltpu.async_copy(s_hbm.at[pl.ds(off_of(c), C)], idx_s[b], isem[b])
            pltpu.async_copy(e_hbm.at[pl.ds(off_of(c), C)], idx_e[b], isem[b])

        def wait_idx(b):
            pltpu.make_async_copy(s_hbm.at[pl.ds(0, C)], idx_s[b], isem[b]).wait()
            pltpu.make_async_copy(e_hbm.at[pl.ds(0, C)], idx_e[b], isem[b]).wait()

        def issue_gather(b):
            pltpu.async_copy(a_hbm.at[idx_s[b]], buf_a[b], gsem[b])
            pltpu.async_copy(b_hbm.at[idx_e[b]], buf_b[b], gsem[b])

        def wait_gather(b):
            pltpu.make_async_copy(a_hbm.at[idx_s[b]], buf_a[b], gsem[b]).wait()
            pltpu.make_async_copy(b_hbm.at[idx_e[b]], buf_b[b], gsem[b]).wait()

        def wait_store(b):
            pltpu.make_async_copy(buf_o[b], out_hbm.at[pl.ds(0, C)], osem[b]).wait()

        issue_idx(0, 0)
        issue_idx(1, 1)
        wait_idx(0)
        issue_gather(0)

        def step(kk, carry):
            for b in (0, 1):
                c = 2 * kk + b
                b1 = 1 - b
                wait_gather(b)

                @pl.when(c + 1 < nch)
                def _():
                    wait_idx(b1)
                    issue_gather(b1)

                @pl.when(c + 2 < nch)
                def _():
                    issue_idx(c + 2, b)

                @pl.when(c >= 2)
                def _():
                    wait_store(b)

                def body(j, c2):
                    for l in range(LG):
                        sl = pl.ds(l * 16, 16)
                        buf_o[b][j, sl] = buf_a[b][j, sl] + buf_b[b][j, sl]
                    return c2

                lax.fori_loop(0, C, body, 0)
                pltpu.async_copy(buf_o[b], out_hbm.at[pl.ds(off_of(c), C)],
                                 osem[b])
            return carry

        lax.fori_loop(0, nch // 2, step, 0)
        wait_store(0)
        wait_store(1)

    return k(A, B, start, end)


def _sc_scatter(P, Q, e, start, end):
    E = start.shape[0]
    N = P.shape[0]
    info = plsc.get_sparse_core_info()
    NC, NS = info.num_cores, info.num_subcores
    NW = NC * NS
    Ew = E // NW
    C = 80
    nch = Ew // C
    ZR = 32
    rows_per_tile = (((N + NS - 1) // NS + ZR - 1) // ZR) * ZR
    NP = rows_per_tile * NS
    nz = rows_per_tile // ZR
    mesh = plsc.VectorSubcoreMesh(core_axis_name="c", subcore_axis_name="s")

    nch2 = 2 * ((nch + 1) // 2)

    @functools.partial(
        pl.kernel,
        out_type=jax.ShapeDtypeStruct((NC, NP, H), jnp.float32),
        mesh=mesh,
        scratch_types=[
            [pltpu.VMEM((C,), jnp.int32)] * 2,
            [pltpu.VMEM((C,), jnp.int32)] * 2,
            [pltpu.VMEM((C,), jnp.float32)] * 2,
            [pltpu.VMEM((C, H), jnp.float32)] * 2,
            [pltpu.VMEM((C, H), jnp.float32)] * 2,
            pltpu.VMEM((ZR, H), jnp.float32),
            pltpu.VMEM_SHARED((NP, H), jnp.float32),
            [pltpu.SemaphoreType.DMA] * 2,
            [pltpu.SemaphoreType.DMA] * 2,
        ],
    )
    def k(p_hbm, q_hbm, ev_hbm, s_hbm, e_hbm, out_hbm, idx_s, idx_e, ebuf,
          buf_p, buf_q, zbuf, acc, isem, gsem):
        cid = lax.axis_index("c")
        sid = lax.axis_index("s")
        wid = sid * NC + cid
        base = wid * Ew
        r0 = sid * rows_per_tile

        def issue_idx(c, b):
            off = base + c * C
            pltpu.async_copy(s_hbm.at[pl.ds(off, C)], idx_s[b], isem[b])
            pltpu.async_copy(e_hbm.at[pl.ds(off, C)], idx_e[b], isem[b])
            pltpu.async_copy(ev_hbm.at[pl.ds(off, C)], ebuf[b], isem[b])

        def wait_idx(b):
            pltpu.make_async_copy(s_hbm.at[pl.ds(0, C)], idx_s[b], isem[b]).wait()
            pltpu.make_async_copy(e_hbm.at[pl.ds(0, C)], idx_e[b], isem[b]).wait()
            pltpu.make_async_copy(ev_hbm.at[pl.ds(0, C)], ebuf[b], isem[b]).wait()

        def issue_gather(b):
            pltpu.async_copy(p_hbm.at[idx_s[b]], buf_p[b], gsem[b])
            pltpu.async_copy(q_hbm.at[idx_e[b]], buf_q[b], gsem[b])

        def wait_gather(b):
            pltpu.make_async_copy(p_hbm.at[idx_s[b]], buf_p[b], gsem[b]).wait()
            pltpu.make_async_copy(q_hbm.at[idx_e[b]], buf_q[b], gsem[b]).wait()

        def zb(j, c2):
            for l in range(LG):
                zbuf[j, pl.ds(l * 16, 16)] = jnp.zeros((16,), jnp.float32)
            return c2

        lax.fori_loop(0, ZR, zb, 0)
        for i in range(nz):
            pltpu.sync_copy(zbuf, acc.at[pl.ds(r0 + i * ZR, ZR)])
        plsc.subcore_barrier()

        issue_idx(0, 0)
        issue_idx(1, 1)
        wait_idx(0)
        issue_gather(0)

        def step(kk, carry):
            for b in (0, 1):
                c = 2 * kk + b
                b1 = 1 - b

                @pl.when(c < nch)
                def _():
                    wait_gather(b)

                    @pl.when(c + 1 < nch)
                    def _():
                        wait_idx(b1)
                        issue_gather(b1)

                    def body(g, c2):
                        ev = ebuf[b][pl.ds(g * 16, 16)]
                        for l in range(16):
                            bv = jnp.full((16,), ev[l], jnp.float32)
                            j = g * 16 + l
                            for lg in range(LG):
                                sl = pl.ds(lg * 16, 16)
                                buf_p[b][j, sl] = buf_p[b][j, sl] * bv
                                buf_q[b][j, sl] = buf_q[b][j, sl] * bv
                        return c2

                    lax.fori_loop(0, C // 16, body, 0)
                    pltpu.sync_copy(buf_p[b], acc.at[idx_e[b]], add=True)
                    pltpu.sync_copy(buf_q[b], acc.at[idx_s[b]], add=True)

                    @pl.when(c + 2 < nch)
                    def _():
                        issue_idx(c + 2, b)
            return carry

        lax.fori_loop(0, nch2 // 2, step, 0)
        plsc.subcore_barrier()
        for i in range(nz):
            rr = r0 + i * ZR
            pltpu.sync_copy(acc.at[pl.ds(rr, ZR)], out_hbm.at[cid, pl.ds(rr, ZR)])

    return k(P, Q, e, start, end)[:, :N, :]


def kernel(x, edge_index, batch, params):
    N = x.shape[0]
    E = edge_index.shape[1]
    start = edge_index[0]
    end = edge_index[1]
    O = (batch[:, None] == jnp.arange(G, dtype=batch.dtype)[None, :]).astype(
        jnp.float32)

    RN = 1000
    NB = N // RN
    RE = 1000
    EB = E // RE

    (wi, bi), = params["input"]
    (w1, b1), (w2, b2), (w3, b3), (w4, b4) = params["edge"]
    (wn1, bn1), (wn2, bn2), (wn3, bn3), (wn4, bn4) = params["node"]
    (wv1, bv1), (wv2, bv2), (wv3, bv3) = params["vp"]
    (wp1, bp1), (wp2, bp2), (wp3, bp3) = params["ip"]

    w1a, w1b, w1c = w1[:H], w1[H:2 * H], w1[2 * H:]
    wna, wnb, wnc, wnd = wn1[:H], wn1[H:2 * H], wn1[2 * H:3 * H], wn1[3 * H:]
    row = lambda v: v.reshape(1, -1)
    bi_, b1_, bn1_ = row(bi), row(b1), row(bn1)
    b2_, b3_, b4_ = row(b2), row(b3), row(b4)
    bn2_, bn3_, bn4_ = row(bn2), row(bn3), row(bn4)
    bv1_, bv2_, bv3_ = row(bv1), row(bv2), row(bv3)
    bp1_, bp2_, bp3_ = row(bp1), row(bp2), row(bp3)
    w4r = w4.reshape(1, H)

    wspec = _full((H, H))
    bspec = _full((1, H))
    gspec = _full((G, H))

    x1, ps, cnt = pl.pallas_call(
        _input_pool_kernel,
        grid=(NB,),
        in_specs=[_rows(RN, H), _rows(RN, G), wspec, bspec],
        out_specs=[_rows(RN, H), gspec, gspec],
        out_shape=[
            jax.ShapeDtypeStruct((N, H), jnp.float32),
            jax.ShapeDtypeStruct((G, H), jnp.float32),
            jax.ShapeDtypeStruct((G, H), jnp.float32),
        ],
    )(x, O, wi, bi_)

    w2b = w2.astype(jnp.bfloat16)
    w3b = w3.astype(jnp.bfloat16)
    vp_all = []

    xc = x1
    e_col = None
    for _ in range(3):
        vp, cv, vd, A, B, P, Q, Rt = pl.pallas_call(
            _tables_kernel,
            grid=(NB,),
            in_specs=[_rows(RN, H), _rows(RN, G), gspec, gspec] +
                     [wspec, bspec] * 3 + [wspec, wspec] +
                     [wspec, wspec, bspec, wspec, wspec, wspec, bspec],
            out_specs=[gspec] * 3 + [_rows(RN, H)] * 5,
            out_shape=[jax.ShapeDtypeStruct((G, H), jnp.float32)] * 3 +
                      [jax.ShapeDtypeStruct((N, H), jnp.float32)] * 5,
        )(xc, O, ps, cnt, wv1, bv1_, wv2, bv2_, wv3, bv3_, w1c, wnd,
          w1a, w1b, b1_, wna, wnb, wnc, bn1_)
        vp_all.append(vp)

        h1 = _sc_gather(A, B, start, end)

        e_col = pl.pallas_call(
            _edge_tail_kernel,
            grid=(EB,),
            in_specs=[_rows(RE, H), _full((H, H)), bspec, _full((H, H)),
                      bspec, bspec, _full((1, 1))],
            out_specs=_rows(RE, 1),
            out_shape=jax.ShapeDtypeStruct((E, 1), jnp.float32),
        )(h1, w2b, b2_, w3b, b3_, w4r, b4_)

        partials = _sc_scatter(P, Q, e_col.reshape(E), start, end)

        xc, ps = pl.pallas_call(
            _node_kernel,
            grid=(NB,),
            in_specs=[_rows(RN, H)] * 4 + [_rows(RN, G)] +
                     [wspec, bspec] * 3,
            out_specs=[_rows(RN, H), gspec],
            out_shape=[
                jax.ShapeDtypeStruct((N, H), jnp.float32),
                jax.ShapeDtypeStruct((G, H), jnp.float32),
            ],
        )(partials[0], partials[1], Rt, xc, O, wn2, bn2_, wn3, bn3_, wn4,
          bn4_)

    ip = pl.pallas_call(
        _ip_kernel,
        grid=(1,),
        in_specs=[gspec] * 5 + [wspec, bspec] * 3 +
                 [_full((4 * H, H)), bspec] + [wspec, bspec] * 2,
        out_specs=gspec,
        out_shape=jax.ShapeDtypeStruct((G, H), jnp.float32),
    )(vp_all[0], vp_all[1], vp_all[2], ps, cnt, wv1, bv1_, wv2, bv2_, wv3,
      bv3_, wp1, bp1_, wp2, bp2_, wp3, bp3_)

    return (e_col.reshape(E), xc, ip)


# speedup vs baseline: 5.1955x; 1.0323x over previous
import functools

import jax
import jax.numpy as jnp
from jax import lax
from jax.experimental import pallas as pl
from jax.experimental.pallas import tpu as pltpu
from jax.experimental.pallas import tpu_sc as plsc

H = 128
G = 64
LG = H // 16


def _ln(x):
    m = jnp.mean(x, axis=-1, keepdims=True)
    v = jnp.mean((x - m) ** 2, axis=-1, keepdims=True)
    return (x - m) * lax.rsqrt(v + 1e-5)


def _dot(a, b):
    return jnp.dot(a, b, preferred_element_type=jnp.float32)


def _input_pool_kernel(x_ref, o_ref, wi_ref, bi_ref, x1_ref, ps_ref, cnt_ref):
    x1 = jnp.tanh(_ln(_dot(x_ref[...], wi_ref[...]) + bi_ref[...]))
    x1_ref[...] = x1
    ot = o_ref[...]

    @pl.when(pl.program_id(0) == 0)
    def _():
        ps_ref[...] = jnp.zeros_like(ps_ref)
        cnt_ref[...] = jnp.zeros_like(cnt_ref)

    ps_ref[...] += lax.dot_general(ot, x1, (((0,), (0,)), ((), ())),
                                   preferred_element_type=jnp.float32)
    cnt_ref[...] += lax.dot_general(
        ot, jnp.ones_like(x1), (((0,), (0,)), ((), ())),
        preferred_element_type=jnp.float32)


def _vp_mlp(ps, cnt, wv1, bv1, wv2, bv2, wv3, bv3):
    h = ps / cnt
    h = jnp.tanh(_ln(_dot(h, wv1[...]) + bv1[...]))
    h = jnp.tanh(_ln(_dot(h, wv2[...]) + bv2[...]))
    return jnp.tanh(_ln(_dot(h, wv3[...]) + bv3[...]))


def _tables_kernel(x_ref, o_ref, ps_ref, cnt_ref, wv1, bv1, wv2, bv2, wv3,
                   bv3, w1c, wnd, w1a, w1b, b1, wna, wnb, wnc, bn,
                   vp_ref, cv_ref, vd_ref, a_ref, b_ref, p_ref, q_ref, r_ref):
    @pl.when(pl.program_id(0) == 0)
    def _():
        vp = _vp_mlp(ps_ref[...], cnt_ref[...], wv1, bv1, wv2, bv2, wv3, bv3)
        vp_ref[...] = vp
        cv_ref[...] = _dot(vp, w1c[...])
        vd_ref[...] = _dot(vp, wnd[...])

    x = x_ref[...]
    o = o_ref[...]
    a_ref[...] = _dot(x, w1a[...]) + _dot(o, cv_ref[...]) + b1[...]
    b_ref[...] = _dot(x, w1b[...])
    p_ref[...] = _dot(x, wna[...])
    q_ref[...] = _dot(x, wnb[...])
    r_ref[...] = _dot(x, wnc[...]) + _dot(o, vd_ref[...]) + bn[...]


def _edge_tail_kernel(h1_ref, w2, b2, w3, b3, w4r, b4, e_ref):
    u = jnp.tanh(_ln(h1_ref[...]))
    u = _dot(u.astype(jnp.bfloat16), w2[...]) + b2[...]
    u = jnp.tanh(_ln(u))
    u = _dot(u.astype(jnp.bfloat16), w3[...]) + b3[...]
    u = jnp.tanh(_ln(u))
    logit = jnp.sum(u * w4r[...], axis=-1, keepdims=True) + b4[...]
    e_ref[...] = 1.0 / (1.0 + jnp.exp(-logit))


def _node_kernel(agga_ref, aggb_ref, aggc_ref, aggd_ref, rt_ref, xin_ref,
                 o_ref, wn2, bn2, wn3, bn3, wn4, bn4, xo_ref, ps_ref):
    h = jnp.tanh(_ln(agga_ref[...] + aggb_ref[...] + aggc_ref[...] +
                     aggd_ref[...] + rt_ref[...]))
    h = jnp.tanh(_ln(_dot(h, wn2[...]) + bn2[...]))
    h = jnp.tanh(_ln(_dot(h, wn3[...]) + bn3[...]))
    h = jnp.tanh(_ln(_dot(h, wn4[...]) + bn4[...]))

    @pl.when(pl.program_id(0) == 0)
    def _():
        ps_ref[...] = jnp.zeros_like(ps_ref)

    ps_ref[...] += lax.dot_general(o_ref[...], h, (((0,), (0,)), ((), ())),
                                   preferred_element_type=jnp.float32)
    xo_ref[...] = h + xin_ref[...]


def _ip_kernel(v0, v1, v2, ps_ref, cnt_ref, wv1, bv1, wv2, bv2, wv3, bv3,
               wp1, bp1, wp2, bp2, wp3, bp3, out_ref):
    v3 = _vp_mlp(ps_ref[...], cnt_ref[...], wv1, bv1, wv2, bv2, wv3, bv3)
    h = jnp.concatenate([v0[...], v1[...], v2[...], v3], axis=1)
    h = jnp.tanh(_ln(_dot(h, wp1[...]) + bp1[...]))
    h = jnp.tanh(_ln(_dot(h, wp2[...]) + bp2[...]))
    h = jnp.tanh(_ln(_dot(h, wp3[...]) + bp3[...]))
    out_ref[...] = h


def _full(shape):
    return pl.BlockSpec(shape, lambda i: (0,) * len(shape))


def _rows(bs, width):
    return pl.BlockSpec((bs, width), lambda i: (i, 0))


def _sc_gather(A, B, start, end):
    E = start.shape[0]
    info = plsc.get_sparse_core_info()
    NW = info.num_cores * info.num_subcores
    Ew = E // NW
    C = 128
    nch = 2 * ((Ew + 2 * C - 1) // (2 * C))
    mesh = plsc.VectorSubcoreMesh(core_axis_name="c", subcore_axis_name="s")

    @functools.partial(
        pl.kernel,
        out_type=jax.ShapeDtypeStruct((E, H), jnp.float32),
        mesh=mesh,
        scratch_types=[
            [pltpu.VMEM((C,), jnp.int32)] * 2,
            [pltpu.VMEM((C,), jnp.int32)] * 2,
            [pltpu.VMEM((C, H), jnp.float32)] * 2,
            [pltpu.VMEM((C, H), jnp.float32)] * 2,
            [pltpu.VMEM((C, H), jnp.float32)] * 2,
            [pltpu.SemaphoreType.DMA] * 2,
            [pltpu.SemaphoreType.DMA] * 2,
            [pltpu.SemaphoreType.DMA] * 2,
        ],
    )
    def k(a_hbm, b_hbm, s_hbm, e_hbm, out_hbm, idx_s, idx_e, buf_a, buf_b,
          buf_o, isem, gsem, osem):
        wid = lax.axis_index("s") * info.num_cores + lax.axis_index("c")
        base = wid * Ew
        off_of = lambda c: base + jnp.minimum(c * C, Ew - C)

        def issue_idx(c, b):
            pltpu.async_copy(s_hbm.at[pl.ds(off_of(c), C)], idx_s[b], isem[b])
            pltpu.async_copy(e_hbm.at[pl.ds(off_of(c), C)], idx_e[b], isem[b])

        def wait_idx(b):
            pltpu.make_async_copy(s_hbm.at[pl.ds(0, C)], idx_s[b], isem[b]).wait()
            pltpu.make_async_copy(e_hbm.at[pl.ds(0, C)], idx_e[b], isem[b]).wait()

        def issue_gather(b):
            pltpu.async_copy(a_hbm.at[idx_s[b]], buf_a[b], gsem[b])
            pltpu.async_copy(b_hbm.at[idx_e[b]], buf_b[b], gsem[b])

        def wait_gather(b):
            pltpu.make_async_copy(a_hbm.at[idx_s[b]], buf_a[b], gsem[b]).wait()
            pltpu.make_async_copy(b_hbm.at[idx_e[b]], buf_b[b], gsem[b]).wait()

        def wait_store(b):
            pltpu.make_async_copy(buf_o[b], out_hbm.at[pl.ds(0, C)], osem[b]).wait()

        issue_idx(0, 0)
        issue_idx(1, 1)
        wait_idx(0)
        issue_gather(0)

        def step(kk, carry):
            for b in (0, 1):
                c = 2 * kk + b
                b1 = 1 - b
                wait_gather(b)

                @pl.when(c + 1 < nch)
                def _():
                    wait_idx(b1)
                    issue_gather(b1)

                @pl.when(c + 2 < nch)
                def _():
                    issue_idx(c + 2, b)

                @pl.when(c >= 2)
                def _():
                    wait_store(b)

                def body(j, c2):
                    for l in range(LG):
                        sl = pl.ds(l * 16, 16)
                        buf_o[b][j, sl] = buf_a[b][j, sl] + buf_b[b][j, sl]
                    return c2

                lax.fori_loop(0, C, body, 0)
                pltpu.async_copy(buf_o[b], out_hbm.at[pl.ds(off_of(c), C)],
                                 osem[b])
            return carry

        lax.fori_loop(0, nch // 2, step, 0)
        wait_store(0)
        wait_store(1)

    return k(A, B, start, end)


def _sc_scatter(P, Q, e, start, end):
    E = start.shape[0]
    N = P.shape[0]
    info = plsc.get_sparse_core_info()
    NC, NS = info.num_cores, info.num_subcores
    NW = NC * NS
    Ew = E // NW
    C = 80
    nch = Ew // C
    ZR = 32
    rows_per_tile = (((N + NS - 1) // NS + ZR - 1) // ZR) * ZR
    NP = rows_per_tile * NS
    nz = rows_per_tile // ZR
    mesh = plsc.VectorSubcoreMesh(core_axis_name="c", subcore_axis_name="s")

    nch2 = 2 * ((nch + 1) // 2)

    @functools.partial(
        pl.kernel,
        out_type=jax.ShapeDtypeStruct((NC, NP, H), jnp.float32),
        mesh=mesh,
        scratch_types=[
            [pltpu.VMEM((C,), jnp.int32)] * 2,
            [pltpu.VMEM((C,), jnp.int32)] * 2,
            [pltpu.VMEM((C,), jnp.float32)] * 2,
            [pltpu.VMEM((C, H), jnp.float32)] * 2,
            [pltpu.VMEM((C, H), jnp.float32)] * 2,
            pltpu.VMEM((ZR, H), jnp.float32),
            pltpu.VMEM_SHARED((NP, H), jnp.float32),
            [pltpu.SemaphoreType.DMA] * 2,
            [pltpu.SemaphoreType.DMA] * 2,
        ],
    )
    def k(p_hbm, q_hbm, ev_hbm, s_hbm, e_hbm, out_hbm, idx_s, idx_e, ebuf,
          buf_p, buf_q, zbuf, acc, isem, gsem):
        cid = lax.axis_index("c")
        sid = lax.axis_index("s")
        wid = sid * NC + cid
        base = wid * Ew
        r0 = sid * rows_per_tile

        def issue_idx(c, b):
            off = base + c * C
            pltpu.async_copy(s_hbm.at[pl.ds(off, C)], idx_s[b], isem[b])
            pltpu.async_copy(e_hbm.at[pl.ds(off, C)], idx_e[b], isem[b])
            pltpu.async_copy(ev_hbm.at[pl.ds(off, C)], ebuf[b], isem[b])

        def wait_idx(b):
            pltpu.make_async_copy(s_hbm.at[pl.ds(0, C)], idx_s[b], isem[b]).wait()
            pltpu.make_async_copy(e_hbm.at[pl.ds(0, C)], idx_e[b], isem[b]).wait()
            pltpu.make_async_copy(ev_hbm.at[pl.ds(0, C)], ebuf[b], isem[b]).wait()

        def issue_gather(b):
            pltpu.async_copy(p_hbm.at[idx_s[b]], buf_p[b], gsem[b])
            pltpu.async_copy(q_hbm.at[idx_e[b]], buf_q[b], gsem[b])

        def wait_gather(b):
            pltpu.make_async_copy(p_hbm.at[idx_s[b]], buf_p[b], gsem[b]).wait()
            pltpu.make_async_copy(q_hbm.at[idx_e[b]], buf_q[b], gsem[b]).wait()

        def zb(j, c2):
            for l in range(LG):
                zbuf[j, pl.ds(l * 16, 16)] = jnp.zeros((16,), jnp.float32)
            return c2

        lax.fori_loop(0, ZR, zb, 0)
        for i in range(nz):
            pltpu.sync_copy(zbuf, acc.at[pl.ds(r0 + i * ZR, ZR)])
        plsc.subcore_barrier()

        issue_idx(0, 0)
        issue_idx(1, 1)
        wait_idx(0)
        issue_gather(0)

        def step(kk, carry):
            for b in (0, 1):
                c = 2 * kk + b
                b1 = 1 - b

                @pl.when(c < nch)
                def _():
                    wait_gather(b)

                    @pl.when(c + 1 < nch)
                    def _():
                        wait_idx(b1)
                        issue_gather(b1)

                    def body(g, c2):
                        ev = ebuf[b][pl.ds(g * 16, 16)]
                        for l in range(16):
                            bv = jnp.full((16,), ev[l], jnp.float32)
                            j = g * 16 + l
                            for lg in range(LG):
                                sl = pl.ds(lg * 16, 16)
                                buf_p[b][j, sl] = buf_p[b][j, sl] * bv
                                buf_q[b][j, sl] = buf_q[b][j, sl] * bv
                        return c2

                    lax.fori_loop(0, C // 16, body, 0)
                    pltpu.sync_copy(buf_p[b], acc.at[idx_e[b]], add=True)
                    pltpu.sync_copy(buf_q[b], acc.at[idx_s[b]], add=True)

                    @pl.when(c + 2 < nch)
                    def _():
                        issue_idx(c + 2, b)
            return carry

        lax.fori_loop(0, nch2 // 2, step, 0)
        plsc.subcore_barrier()
        for i in range(nz):
            rr = r0 + i * ZR
            pltpu.sync_copy(acc.at[pl.ds(rr, ZR)], out_hbm.at[cid, pl.ds(rr, ZR)])

    return k(P, Q, e, start, end)[:, :N, :]


def kernel(x, edge_index, batch, params):
    N = x.shape[0]
    E = edge_index.shape[1]
    start = edge_index[0]
    end = edge_index[1]
    O = (batch[:, None] == jnp.arange(G, dtype=batch.dtype)[None, :]).astype(
        jnp.float32)

    RN = 1000
    NB = N // RN
    RE = 640
    E1 = (E // (2 * 2560) + 1) * 2560
    spans = [(0, E1), (E1, E - E1)]

    (wi, bi), = params["input"]
    (w1, b1), (w2, b2), (w3, b3), (w4, b4) = params["edge"]
    (wn1, bn1), (wn2, bn2), (wn3, bn3), (wn4, bn4) = params["node"]
    (wv1, bv1), (wv2, bv2), (wv3, bv3) = params["vp"]
    (wp1, bp1), (wp2, bp2), (wp3, bp3) = params["ip"]

    w1a, w1b, w1c = w1[:H], w1[H:2 * H], w1[2 * H:]
    wna, wnb, wnc, wnd = wn1[:H], wn1[H:2 * H], wn1[2 * H:3 * H], wn1[3 * H:]
    row = lambda v: v.reshape(1, -1)
    bi_, b1_, bn1_ = row(bi), row(b1), row(bn1)
    b2_, b3_, b4_ = row(b2), row(b3), row(b4)
    bn2_, bn3_, bn4_ = row(bn2), row(bn3), row(bn4)
    bv1_, bv2_, bv3_ = row(bv1), row(bv2), row(bv3)
    bp1_, bp2_, bp3_ = row(bp1), row(bp2), row(bp3)
    w4r = w4.reshape(1, H)

    wspec = _full((H, H))
    bspec = _full((1, H))
    gspec = _full((G, H))

    x1, ps, cnt = pl.pallas_call(
        _input_pool_kernel,
        grid=(NB,),
        in_specs=[_rows(RN, H), _rows(RN, G), wspec, bspec],
        out_specs=[_rows(RN, H), gspec, gspec],
        out_shape=[
            jax.ShapeDtypeStruct((N, H), jnp.float32),
            jax.ShapeDtypeStruct((G, H), jnp.float32),
            jax.ShapeDtypeStruct((G, H), jnp.float32),
        ],
    )(x, O, wi, bi_)

    w2b = w2.astype(jnp.bfloat16)
    w3b = w3.astype(jnp.bfloat16)
    vp_all = []
    s_h = [lax.slice_in_dim(start, o, o + l) for o, l in spans]
    e_h = [lax.slice_in_dim(end, o, o + l) for o, l in spans]

    xc = x1
    e_parts = None
    for _ in range(3):
        vp, cv, vd, A, B, P, Q, Rt = pl.pallas_call(
            _tables_kernel,
            grid=(NB,),
            in_specs=[_rows(RN, H), _rows(RN, G), gspec, gspec] +
                     [wspec, bspec] * 3 + [wspec, wspec] +
                     [wspec, wspec, bspec, wspec, wspec, wspec, bspec],
            out_specs=[gspec] * 3 + [_rows(RN, H)] * 5,
            out_shape=[jax.ShapeDtypeStruct((G, H), jnp.float32)] * 3 +
                      [jax.ShapeDtypeStruct((N, H), jnp.float32)] * 5,
        )(xc, O, ps, cnt, wv1, bv1_, wv2, bv2_, wv3, bv3_, w1c, wnd,
          w1a, w1b, b1_, wna, wnb, wnc, bn1_)
        vp_all.append(vp)

        h1s = [_sc_gather(A, B, s_h[i], e_h[i]) for i in range(2)]

        e_parts = [
            pl.pallas_call(
                _edge_tail_kernel,
                grid=(spans[i][1] // RE,),
                in_specs=[_rows(RE, H), _full((H, H)), bspec, _full((H, H)),
                          bspec, bspec, _full((1, 1))],
                out_specs=_rows(RE, 1),
                out_shape=jax.ShapeDtypeStruct((spans[i][1], 1), jnp.float32),
            )(h1s[i], w2b, b2_, w3b, b3_, w4r, b4_)
            for i in range(2)
        ]

        parts = [
            _sc_scatter(P, Q, e_parts[i].reshape(spans[i][1]), s_h[i], e_h[i])
            for i in range(2)
        ]

        xc, ps = pl.pallas_call(
            _node_kernel,
            grid=(NB,),
            in_specs=[_rows(RN, H)] * 6 + [_rows(RN, G)] +
                     [wspec, bspec] * 3,
            out_specs=[_rows(RN, H), gspec],
            out_shape=[
                jax.ShapeDtypeStruct((N, H), jnp.float32),
                jax.ShapeDtypeStruct((G, H), jnp.float32),
            ],
        )(parts[0][0], parts[0][1], parts[1][0], parts[1][1], Rt, xc, O,
          wn2, bn2_, wn3, bn3_, wn4, bn4_)

    ip = pl.pallas_call(
        _ip_kernel,
        grid=(1,),
        in_specs=[gspec] * 5 + [wspec, bspec] * 3 +
                 [_full((4 * H, H)), bspec] + [wspec, bspec] * 2,
        out_specs=gspec,
        out_shape=jax.ShapeDtypeStruct((G, H), jnp.float32),
    )(vp_all[0], vp_all[1], vp_all[2], ps, cnt, wv1, bv1_, wv2, bv2_, wv3,
      bv3_, wp1, bp1_, wp2, bp2_, wp3, bp3_)

    e_out = jnp.concatenate(
        [e_parts[0].reshape(spans[0][1]), e_parts[1].reshape(spans[1][1])])
    return (e_out, xc, ip)


# speedup vs baseline: 5.1964x; 1.0002x over previous
import functools

import jax
import jax.numpy as jnp
from jax import lax
from jax.experimental import pallas as pl
from jax.experimental.pallas import tpu as pltpu
from jax.experimental.pallas import tpu_sc as plsc

H = 128
G = 64
LG = H // 16


def _ln(x):
    m = jnp.mean(x, axis=-1, keepdims=True)
    v = jnp.mean((x - m) ** 2, axis=-1, keepdims=True)
    return (x - m) * lax.rsqrt(v + 1e-5)


def _dot(a, b):
    return jnp.dot(a, b, preferred_element_type=jnp.float32)


def _input_pool_kernel(x_ref, o_ref, wi_ref, bi_ref, x1_ref, ps_ref, cnt_ref):
    x1 = jnp.tanh(_ln(_dot(x_ref[...], wi_ref[...]) + bi_ref[...]))
    x1_ref[...] = x1
    ot = o_ref[...]

    @pl.when(pl.program_id(0) == 0)
    def _():
        ps_ref[...] = jnp.zeros_like(ps_ref)
        cnt_ref[...] = jnp.zeros_like(cnt_ref)

    ps_ref[...] += lax.dot_general(ot, x1, (((0,), (0,)), ((), ())),
                                   preferred_element_type=jnp.float32)
    cnt_ref[...] += lax.dot_general(
        ot, jnp.ones_like(x1), (((0,), (0,)), ((), ())),
        preferred_element_type=jnp.float32)


def _vp_mlp(ps, cnt, wv1, bv1, wv2, bv2, wv3, bv3):
    h = ps / cnt
    h = jnp.tanh(_ln(_dot(h, wv1[...]) + bv1[...]))
    h = jnp.tanh(_ln(_dot(h, wv2[...]) + bv2[...]))
    return jnp.tanh(_ln(_dot(h, wv3[...]) + bv3[...]))


def _tables_kernel(x_ref, o_ref, ps_ref, cnt_ref, wv1, bv1, wv2, bv2, wv3,
                   bv3, w1c, wnd, w1a, w1b, b1, wna, wnb, wnc, bn,
                   vp_ref, cv_ref, vd_ref, a_ref, b_ref, p_ref, q_ref, r_ref):
    @pl.when(pl.program_id(0) == 0)
    def _():
        vp = _vp_mlp(ps_ref[...], cnt_ref[...], wv1, bv1, wv2, bv2, wv3, bv3)
        vp_ref[...] = vp
        cv_ref[...] = _dot(vp, w1c[...])
        vd_ref[...] = _dot(vp, wnd[...])

    x = x_ref[...]
    o = o_ref[...]
    a_ref[...] = _dot(x, w1a[...]) + _dot(o, cv_ref[...]) + b1[...]
    b_ref[...] = _dot(x, w1b[...])
    p_ref[...] = _dot(x, wna[...])
    q_ref[...] = _dot(x, wnb[...])
    r_ref[...] = _dot(x, wnc[...]) + _dot(o, vd_ref[...]) + bn[...]


def _edge_tail_kernel(h1_ref, w2, b2, w3, b3, w4r, b4, e_ref):
    u = jnp.tanh(_ln(h1_ref[...]))
    u = _dot(u.astype(jnp.bfloat16), w2[...]) + b2[...]
    u = jnp.tanh(_ln(u))
    u = _dot(u.astype(jnp.bfloat16), w3[...]) + b3[...]
    u = jnp.tanh(_ln(u))
    logit = jnp.sum(u * w4r[...], axis=-1, keepdims=True) + b4[...]
    e_ref[...] = 1.0 / (1.0 + jnp.exp(-logit))


def _node_kernel(agga_ref, aggb_ref, aggc_ref, aggd_ref, rt_ref, xin_ref,
                 o_ref, wn2, bn2, wn3, bn3, wn4, bn4, xo_ref, ps_ref):
    h = jnp.tanh(_ln(agga_ref[...] + aggb_ref[...] + aggc_ref[...] +
                     aggd_ref[...] + rt_ref[...]))
    h = jnp.tanh(_ln(_dot(h, wn2[...]) + bn2[...]))
    h = jnp.tanh(_ln(_dot(h, wn3[...]) + bn3[...]))
    h = jnp.tanh(_ln(_dot(h, wn4[...]) + bn4[...]))

    @pl.when(pl.program_id(0) == 0)
    def _():
        ps_ref[...] = jnp.zeros_like(ps_ref)

    ps_ref[...] += lax.dot_general(o_ref[...], h, (((0,), (0,)), ((), ())),
                                   preferred_element_type=jnp.float32)
    xo_ref[...] = h + xin_ref[...]


def _ip_kernel(v0, v1, v2, ps_ref, cnt_ref, wv1, bv1, wv2, bv2, wv3, bv3,
               wp1, bp1, wp2, bp2, wp3, bp3, out_ref):
    v3 = _vp_mlp(ps_ref[...], cnt_ref[...], wv1, bv1, wv2, bv2, wv3, bv3)
    h = jnp.concatenate([v0[...], v1[...], v2[...], v3], axis=1)
    h = jnp.tanh(_ln(_dot(h, wp1[...]) + bp1[...]))
    h = jnp.tanh(_ln(_dot(h, wp2[...]) + bp2[...]))
    h = jnp.tanh(_ln(_dot(h, wp3[...]) + bp3[...]))
    out_ref[...] = h


def _full(shape):
    return pl.BlockSpec(shape, lambda i: (0,) * len(shape))


def _rows(bs, width):
    return pl.BlockSpec((bs, width), lambda i: (i, 0))


def _sc_gather(A, B, start, end):
    E = start.shape[0]
    info = plsc.get_sparse_core_info()
    NW = info.num_cores * info.num_subcores
    Ew = E // NW
    C = 128
    nch = 2 * ((Ew + 2 * C - 1) // (2 * C))
    mesh = plsc.VectorSubcoreMesh(core_axis_name="c", subcore_axis_name="s")

    @functools.partial(
        pl.kernel,
        out_type=jax.ShapeDtypeStruct((E, H), jnp.float32),
        mesh=mesh,
        cost_estimate=pl.CostEstimate(
            flops=E * H, transcendentals=0,
            bytes_accessed=3 * E * H * 4 + 2 * E * 4),
        scratch_types=[
            [pltpu.VMEM((C,), jnp.int32)] * 2,
            [pltpu.VMEM((C,), jnp.int32)] * 2,
            [pltpu.VMEM((C, H), jnp.float32)] * 2,
            [pltpu.VMEM((C, H), jnp.float32)] * 2,
            [pltpu.VMEM((C, H), jnp.float32)] * 2,
            [pltpu.SemaphoreType.DMA] * 2,
            [pltpu.SemaphoreType.DMA] * 2,
            [pltpu.SemaphoreType.DMA] * 2,
        ],
    )
    def k(a_hbm, b_hbm, s_hbm, e_hbm, out_hbm, idx_s, idx_e, buf_a, buf_b,
          buf_o, isem, gsem, osem):
        wid = lax.axis_index("s") * info.num_cores + lax.axis_index("c")
        base = wid * Ew
        off_of = lambda c: base + jnp.minimum(c * C, Ew - C)

        def issue_idx(c, b):
            pltpu.async_copy(s_hbm.at[pl.ds(off_of(c), C)], idx_s[b], isem[b])
            pltpu.async_copy(e_hbm.at[pl.ds(off_of(c), C)], idx_e[b], isem[b])

        def wait_idx(b):
            pltpu.make_async_copy(s_hbm.at[pl.ds(0, C)], idx_s[b], isem[b]).wait()
            pltpu.make_async_copy(e_hbm.at[pl.ds(0, C)], idx_e[b], isem[b]).wait()

        def issue_gather(b):
            pltpu.async_copy(a_hbm.at[idx_s[b]], buf_a[b], gsem[b])
            pltpu.async_copy(b_hbm.at[idx_e[b]], buf_b[b], gsem[b])

        def wait_gather(b):
            pltpu.make_async_copy(a_hbm.at[idx_s[b]], buf_a[b], gsem[b]).wait()
            pltpu.make_async_copy(b_hbm.at[idx_e[b]], buf_b[b], gsem[b]).wait()

        def wait_store(b):
            pltpu.make_async_copy(buf_o[b], out_hbm.at[pl.ds(0, C)], osem[b]).wait()

        issue_idx(0, 0)
        issue_idx(1, 1)
        wait_idx(0)
        issue_gather(0)

        def step(kk, carry):
            for b in (0, 1):
                c = 2 * kk + b
                b1 = 1 - b
                wait_gather(b)

                @pl.when(c + 1 < nch)
                def _():
                    wait_idx(b1)
                    issue_gather(b1)

                @pl.when(c + 2 < nch)
                def _():
                    issue_idx(c + 2, b)

                @pl.when(c >= 2)
                def _():
                    wait_store(b)

                def body(j, c2):
                    for l in range(LG):
                        sl = pl.ds(l * 16, 16)
                        buf_o[b][j, sl] = buf_a[b][j, sl] + buf_b[b][j, sl]
                    return c2

                lax.fori_loop(0, C, body, 0)
                pltpu.async_copy(buf_o[b], out_hbm.at[pl.ds(off_of(c), C)],
                                 osem[b])
            return carry

        lax.fori_loop(0, nch // 2, step, 0)
        wait_store(0)
        wait_store(1)

    return k(A, B, start, end)


def _sc_scatter(P, Q, e, start, end):
    E = start.shape[0]
    N = P.shape[0]
    info = plsc.get_sparse_core_info()
    NC, NS = info.num_cores, info.num_subcores
    NW = NC * NS
    Ew = E // NW
    C = 80
    nch = Ew // C
    ZR = 32
    rows_per_tile = (((N + NS - 1) // NS + ZR - 1) // ZR) * ZR
    NP = rows_per_tile * NS
    nz = rows_per_tile // ZR
    mesh = plsc.VectorSubcoreMesh(core_axis_name="c", subcore_axis_name="s")

    nch2 = 2 * ((nch + 1) // 2)

    @functools.partial(
        pl.kernel,
        out_type=jax.ShapeDtypeStruct((NC, NP, H), jnp.float32),
        mesh=mesh,
        cost_estimate=pl.CostEstimate(
            flops=3 * E * H, transcendentals=0,
            bytes_accessed=4 * E * H * 4 + 3 * E * 4 + NC * NP * H * 4),
        scratch_types=[
            [pltpu.VMEM((C,), jnp.int32)] * 2,
            [pltpu.VMEM((C,), jnp.int32)] * 2,
            [pltpu.VMEM((C,), jnp.float32)] * 2,
            [pltpu.VMEM((C, H), jnp.float32)] * 2,
            [pltpu.VMEM((C, H), jnp.float32)] * 2,
            pltpu.VMEM((ZR, H), jnp.float32),
            pltpu.VMEM_SHARED((NP, H), jnp.float32),
            [pltpu.SemaphoreType.DMA] * 2,
            [pltpu.SemaphoreType.DMA] * 2,
        ],
    )
    def k(p_hbm, q_hbm, ev_hbm, s_hbm, e_hbm, out_hbm, idx_s, idx_e, ebuf,
          buf_p, buf_q, zbuf, acc, isem, gsem):
        cid = lax.axis_index("c")
        sid = lax.axis_index("s")
        wid = sid * NC + cid
        base = wid * Ew
        r0 = sid * rows_per_tile

        def issue_idx(c, b):
            off = base + c * C
            pltpu.async_copy(s_hbm.at[pl.ds(off, C)], idx_s[b], isem[b])
            pltpu.async_copy(e_hbm.at[pl.ds(off, C)], idx_e[b], isem[b])
            pltpu.async_copy(ev_hbm.at[pl.ds(off, C)], ebuf[b], isem[b])

        def wait_idx(b):
            pltpu.make_async_copy(s_hbm.at[pl.ds(0, C)], idx_s[b], isem[b]).wait()
            pltpu.make_async_copy(e_hbm.at[pl.ds(0, C)], idx_e[b], isem[b]).wait()
            pltpu.make_async_copy(ev_hbm.at[pl.ds(0, C)], ebuf[b], isem[b]).wait()

        def issue_gather(b):
            pltpu.async_copy(p_hbm.at[idx_s[b]], buf_p[b], gsem[b])
            pltpu.async_copy(q_hbm.at[idx_e[b]], buf_q[b], gsem[b])

        def wait_gather(b):
            pltpu.make_async_copy(p_hbm.at[idx_s[b]], buf_p[b], gsem[b]).wait()
            pltpu.make_async_copy(q_hbm.at[idx_e[b]], buf_q[b], gsem[b]).wait()

        def zb(j, c2):
            for l in range(LG):
                zbuf[j, pl.ds(l * 16, 16)] = jnp.zeros((16,), jnp.float32)
            return c2

        lax.fori_loop(0, ZR, zb, 0)
        for i in range(nz):
            pltpu.sync_copy(zbuf, acc.at[pl.ds(r0 + i * ZR, ZR)])
        plsc.subcore_barrier()

        issue_idx(0, 0)
        issue_idx(1, 1)
        wait_idx(0)
        issue_gather(0)

        def step(kk, carry):
            for b in (0, 1):
                c = 2 * kk + b
                b1 = 1 - b

                @pl.when(c < nch)
                def _():
                    wait_gather(b)

                    @pl.when(c + 1 < nch)
                    def _():
                        wait_idx(b1)
                        issue_gather(b1)

                    def body(g, c2):
                        ev = ebuf[b][pl.ds(g * 16, 16)]
                        for l in range(16):
                            bv = jnp.full((16,), ev[l], jnp.float32)
                            j = g * 16 + l
                            for lg in range(LG):
                                sl = pl.ds(lg * 16, 16)
                                buf_p[b][j, sl] = buf_p[b][j, sl] * bv
                                buf_q[b][j, sl] = buf_q[b][j, sl] * bv
                        return c2

                    lax.fori_loop(0, C // 16, body, 0)
                    pltpu.sync_copy(buf_p[b], acc.at[idx_e[b]], add=True)
                    pltpu.sync_copy(buf_q[b], acc.at[idx_s[b]], add=True)

                    @pl.when(c + 2 < nch)
                    def _():
                        issue_idx(c + 2, b)
            return carry

        lax.fori_loop(0, nch2 // 2, step, 0)
        plsc.subcore_barrier()
        for i in range(nz):
            rr = r0 + i * ZR
            pltpu.sync_copy(acc.at[pl.ds(rr, ZR)], out_hbm.at[cid, pl.ds(rr, ZR)])

    return k(P, Q, e, start, end)[:, :N, :]


def kernel(x, edge_index, batch, params):
    N = x.shape[0]
    E = edge_index.shape[1]
    start = edge_index[0]
    end = edge_index[1]
    O = (batch[:, None] == jnp.arange(G, dtype=batch.dtype)[None, :]).astype(
        jnp.float32)

    RN = 1000
    NB = N // RN
    RE = 640
    E1 = (E // (2 * 2560) + 1) * 2560
    spans = [(0, E1), (E1, E - E1)]

    (wi, bi), = params["input"]
    (w1, b1), (w2, b2), (w3, b3), (w4, b4) = params["edge"]
    (wn1, bn1), (wn2, bn2), (wn3, bn3), (wn4, bn4) = params["node"]
    (wv1, bv1), (wv2, bv2), (wv3, bv3) = params["vp"]
    (wp1, bp1), (wp2, bp2), (wp3, bp3) = params["ip"]

    w1a, w1b, w1c = w1[:H], w1[H:2 * H], w1[2 * H:]
    wna, wnb, wnc, wnd = wn1[:H], wn1[H:2 * H], wn1[2 * H:3 * H], wn1[3 * H:]
    row = lambda v: v.reshape(1, -1)
    bi_, b1_, bn1_ = row(bi), row(b1), row(bn1)
    b2_, b3_, b4_ = row(b2), row(b3), row(b4)
    bn2_, bn3_, bn4_ = row(bn2), row(bn3), row(bn4)
    bv1_, bv2_, bv3_ = row(bv1), row(bv2), row(bv3)
    bp1_, bp2_, bp3_ = row(bp1), row(bp2), row(bp3)
    w4r = w4.reshape(1, H)

    wspec = _full((H, H))
    bspec = _full((1, H))
    gspec = _full((G, H))

    x1, ps, cnt = pl.pallas_call(
        _input_pool_kernel,
        grid=(NB,),
        in_specs=[_rows(RN, H), _rows(RN, G), wspec, bspec],
        out_specs=[_rows(RN, H), gspec, gspec],
        out_shape=[
            jax.ShapeDtypeStruct((N, H), jnp.float32),
            jax.ShapeDtypeStruct((G, H), jnp.float32),
            jax.ShapeDtypeStruct((G, H), jnp.float32),
        ],
    )(x, O, wi, bi_)

    w2b = w2.astype(jnp.bfloat16)
    w3b = w3.astype(jnp.bfloat16)
    vp_all = []
    s_h = [lax.slice_in_dim(start, o, o + l) for o, l in spans]
    e_h = [lax.slice_in_dim(end, o, o + l) for o, l in spans]

    xc = x1
    e_parts = None
    for _ in range(3):
        vp, cv, vd, A, B, P, Q, Rt = pl.pallas_call(
            _tables_kernel,
            grid=(NB,),
            in_specs=[_rows(RN, H), _rows(RN, G), gspec, gspec] +
                     [wspec, bspec] * 3 + [wspec, wspec] +
                     [wspec, wspec, bspec, wspec, wspec, wspec, bspec],
            out_specs=[gspec] * 3 + [_rows(RN, H)] * 5,
            out_shape=[jax.ShapeDtypeStruct((G, H), jnp.float32)] * 3 +
                      [jax.ShapeDtypeStruct((N, H), jnp.float32)] * 5,
        )(xc, O, ps, cnt, wv1, bv1_, wv2, bv2_, wv3, bv3_, w1c, wnd,
          w1a, w1b, b1_, wna, wnb, wnc, bn1_)
        vp_all.append(vp)

        h1s = [_sc_gather(A, B, s_h[i], e_h[i]) for i in range(2)]

        e_parts = [
            pl.pallas_call(
                _edge_tail_kernel,
                grid=(spans[i][1] // RE,),
                in_specs=[_rows(RE, H), _full((H, H)), bspec, _full((H, H)),
                          bspec, bspec, _full((1, 1))],
                out_specs=_rows(RE, 1),
                out_shape=jax.ShapeDtypeStruct((spans[i][1], 1), jnp.float32),
            )(h1s[i], w2b, b2_, w3b, b3_, w4r, b4_)
            for i in range(2)
        ]

        parts = [
            _sc_scatter(P, Q, e_parts[i].reshape(spans[i][1]), s_h[i], e_h[i])
            for i in range(2)
        ]

        xc, ps = pl.pallas_call(
            _node_kernel,
            grid=(NB,),
            in_specs=[_rows(RN, H)] * 6 + [_rows(RN, G)] +
                     [wspec, bspec] * 3,
            out_specs=[_rows(RN, H), gspec],
            out_shape=[
                jax.ShapeDtypeStruct((N, H), jnp.float32),
                jax.ShapeDtypeStruct((G, H), jnp.float32),
            ],
        )(parts[0][0], parts[0][1], parts[1][0], parts[1][1], Rt, xc, O,
          wn2, bn2_, wn3, bn3_, wn4, bn4_)

    ip = pl.pallas_call(
        _ip_kernel,
        grid=(1,),
        in_specs=[gspec] * 5 + [wspec, bspec] * 3 +
                 [_full((4 * H, H)), bspec] + [wspec, bspec] * 2,
        out_specs=gspec,
        out_shape=jax.ShapeDtypeStruct((G, H), jnp.float32),
    )(vp_all[0], vp_all[1], vp_all[2], ps, cnt, wv1, bv1_, wv2, bv2_, wv3,
      bv3_, wp1, bp1_, wp2, bp2_, wp3, bp3_)

    e_out = jnp.concatenate(
        [e_parts[0].reshape(spans[0][1]), e_parts[1].reshape(spans[1][1])])
    return (e_out, xc, ip)


# speedup vs baseline: 5.2095x; 1.0025x over previous
import functools

import jax
import jax.numpy as jnp
from jax import lax
from jax.experimental import pallas as pl
from jax.experimental.pallas import tpu as pltpu
from jax.experimental.pallas import tpu_sc as plsc

H = 128
G = 64
LG = H // 16


def _ln(x):
    m = jnp.mean(x, axis=-1, keepdims=True)
    v = jnp.mean((x - m) ** 2, axis=-1, keepdims=True)
    return (x - m) * lax.rsqrt(v + 1e-5)


def _dot(a, b):
    return jnp.dot(a, b, preferred_element_type=jnp.float32)


def _input_pool_kernel(x_ref, o_ref, wi_ref, bi_ref, x1_ref, ps_ref, cnt_ref):
    x1 = jnp.tanh(_ln(_dot(x_ref[...], wi_ref[...]) + bi_ref[...]))
    x1_ref[...] = x1
    ot = o_ref[...]

    @pl.when(pl.program_id(0) == 0)
    def _():
        ps_ref[...] = jnp.zeros_like(ps_ref)
        cnt_ref[...] = jnp.zeros_like(cnt_ref)

    ps_ref[...] += lax.dot_general(ot, x1, (((0,), (0,)), ((), ())),
                                   preferred_element_type=jnp.float32)
    cnt_ref[...] += lax.dot_general(
        ot, jnp.ones_like(x1), (((0,), (0,)), ((), ())),
        preferred_element_type=jnp.float32)


def _vp_mlp(ps, cnt, wv1, bv1, wv2, bv2, wv3, bv3):
    h = ps / cnt
    h = jnp.tanh(_ln(_dot(h, wv1[...]) + bv1[...]))
    h = jnp.tanh(_ln(_dot(h, wv2[...]) + bv2[...]))
    return jnp.tanh(_ln(_dot(h, wv3[...]) + bv3[...]))


def _tables_kernel(x_ref, o_ref, ps_ref, cnt_ref, wv1, bv1, wv2, bv2, wv3,
                   bv3, w1c, wnd, w1a, w1b, b1, wna, wnb, wnc, bn,
                   vp_ref, cv_ref, vd_ref, a_ref, b_ref, p_ref, q_ref, r_ref):
    @pl.when(pl.program_id(0) == 0)
    def _():
        vp = _vp_mlp(ps_ref[...], cnt_ref[...], wv1, bv1, wv2, bv2, wv3, bv3)
        vp_ref[...] = vp
        cv_ref[...] = _dot(vp, w1c[...])
        vd_ref[...] = _dot(vp, wnd[...])

    x = x_ref[...]
    o = o_ref[...]
    a_ref[...] = _dot(x, w1a[...]) + _dot(o, cv_ref[...]) + b1[...]
    b_ref[...] = _dot(x, w1b[...])
    p_ref[...] = _dot(x, wna[...])
    q_ref[...] = _dot(x, wnb[...])
    r_ref[...] = _dot(x, wnc[...]) + _dot(o, vd_ref[...]) + bn[...]


def _edge_tail_kernel(h1_ref, w2, b2, w3, b3, w4r, b4, e_ref):
    u = jnp.tanh(_ln(h1_ref[...].astype(jnp.float32)))
    u = _dot(u.astype(jnp.bfloat16), w2[...]) + b2[...]
    u = jnp.tanh(_ln(u))
    u = _dot(u.astype(jnp.bfloat16), w3[...]) + b3[...]
    u = jnp.tanh(_ln(u))
    logit = jnp.sum(u * w4r[...], axis=-1, keepdims=True) + b4[...]
    e_ref[...] = 1.0 / (1.0 + jnp.exp(-logit))


def _node_kernel(agga_ref, aggb_ref, aggc_ref, aggd_ref, rt_ref, xin_ref,
                 o_ref, wn2, bn2, wn3, bn3, wn4, bn4, xo_ref, ps_ref):
    h = jnp.tanh(_ln(agga_ref[...] + aggb_ref[...] + aggc_ref[...] +
                     aggd_ref[...] + rt_ref[...]))
    h = jnp.tanh(_ln(_dot(h, wn2[...]) + bn2[...]))
    h = jnp.tanh(_ln(_dot(h, wn3[...]) + bn3[...]))
    h = jnp.tanh(_ln(_dot(h, wn4[...]) + bn4[...]))

    @pl.when(pl.program_id(0) == 0)
    def _():
        ps_ref[...] = jnp.zeros_like(ps_ref)

    ps_ref[...] += lax.dot_general(o_ref[...], h, (((0,), (0,)), ((), ())),
                                   preferred_element_type=jnp.float32)
    xo_ref[...] = h + xin_ref[...]


def _ip_kernel(v0, v1, v2, ps_ref, cnt_ref, wv1, bv1, wv2, bv2, wv3, bv3,
               wp1, bp1, wp2, bp2, wp3, bp3, out_ref):
    v3 = _vp_mlp(ps_ref[...], cnt_ref[...], wv1, bv1, wv2, bv2, wv3, bv3)
    h = jnp.concatenate([v0[...], v1[...], v2[...], v3], axis=1)
    h = jnp.tanh(_ln(_dot(h, wp1[...]) + bp1[...]))
    h = jnp.tanh(_ln(_dot(h, wp2[...]) + bp2[...]))
    h = jnp.tanh(_ln(_dot(h, wp3[...]) + bp3[...]))
    out_ref[...] = h


def _full(shape):
    return pl.BlockSpec(shape, lambda i: (0,) * len(shape))


def _rows(bs, width):
    return pl.BlockSpec((bs, width), lambda i: (i, 0))


def _sc_gather(A, B, start, end):
    E = start.shape[0]
    info = plsc.get_sparse_core_info()
    NW = info.num_cores * info.num_subcores
    Ew = E // NW
    C = 128
    nch = 2 * ((Ew + 2 * C - 1) // (2 * C))
    mesh = plsc.VectorSubcoreMesh(core_axis_name="c", subcore_axis_name="s")

    @functools.partial(
        pl.kernel,
        out_type=jax.ShapeDtypeStruct((E, H), jnp.float32),
        mesh=mesh,
        cost_estimate=pl.CostEstimate(
            flops=E * H, transcendentals=0,
            bytes_accessed=3 * E * H * 4 + 2 * E * 4),
        scratch_types=[
            [pltpu.VMEM((C,), jnp.int32)] * 2,
            [pltpu.VMEM((C,), jnp.int32)] * 2,
            [pltpu.VMEM((C, H), jnp.float32)] * 2,
            [pltpu.VMEM((C, H), jnp.float32)] * 2,
            [pltpu.VMEM((C, H), jnp.float32)] * 2,
            [pltpu.SemaphoreType.DMA] * 2,
            [pltpu.SemaphoreType.DMA] * 2,
            [pltpu.SemaphoreType.DMA] * 2,
        ],
    )
    def k(a_hbm, b_hbm, s_hbm, e_hbm, out_hbm, idx_s, idx_e, buf_a, buf_b,
          buf_o, isem, gsem, osem):
        wid = lax.axis_index("s") * info.num_cores + lax.axis_index("c")
        base = wid * Ew
        off_of = lambda c: base + jnp.minimum(c * C, Ew - C)

        def issue_idx(c, b):
            pltpu.async_copy(s_hbm.at[pl.ds(off_of(c), C)], idx_s[b], isem[b])
            pltpu.async_copy(e_hbm.at[pl.ds(off_of(c), C)], idx_e[b], isem[b])

        def wait_idx(b):
            pltpu.make_async_copy(s_hbm.at[pl.ds(0, C)], idx_s[b], isem[b]).wait()
            pltpu.make_async_copy(e_hbm.at[pl.ds(0, C)], idx_e[b], isem[b]).wait()

        def issue_gather(b):
            pltpu.async_copy(a_hbm.at[idx_s[b]], buf_a[b], gsem[b])
            pltpu.async_copy(b_hbm.at[idx_e[b]], buf_b[b], gsem[b])

        def wait_gather(b):
            pltpu.make_async_copy(a_hbm.at[idx_s[b]], buf_a[b], gsem[b]).wait()
            pltpu.make_async_copy(b_hbm.at[idx_e[b]], buf_b[b], gsem[b]).wait()

        def wait_store(b):
            pltpu.make_async_copy(buf_o[b], out_hbm.at[pl.ds(0, C)], osem[b]).wait()

        issue_idx(0, 0)
        issue_idx(1, 1)
        wait_idx(0)
        issue_gather(0)

        def step(kk, carry):
            for b in (0, 1):
                c = 2 * kk + b
                b1 = 1 - b
                wait_gather(b)

                @pl.when(c + 1 < nch)
                def _():
                    wait_idx(b1)
                    issue_gather(b1)

                @pl.when(c + 2 < nch)
                def _():
                    issue_idx(c + 2, b)

                @pl.when(c >= 2)
                def _():
                    wait_store(b)

                def body(j, c2):
                    for l in range(LG):
                        sl = pl.ds(l * 16, 16)
                        buf_o[b][j, sl] = buf_a[b][j, sl] + buf_b[b][j, sl]
                    return c2

                lax.fori_loop(0, C, body, 0)
                pltpu.async_copy(buf_o[b], out_hbm.at[pl.ds(off_of(c), C)],
                                 osem[b])
            return carry

        lax.fori_loop(0, nch // 2, step, 0)
        wait_store(0)
        wait_store(1)

    return k(A, B, start, end)


def _sc_scatter(P, Q, e, start, end):
    E = start.shape[0]
    N = P.shape[0]
    info = plsc.get_sparse_core_info()
    NC, NS = info.num_cores, info.num_subcores
    NW = NC * NS
    Ew = E // NW
    C = 80
    nch = Ew // C
    ZR = 32
    rows_per_tile = (((N + NS - 1) // NS + ZR - 1) // ZR) * ZR
    NP = rows_per_tile * NS
    nz = rows_per_tile // ZR
    mesh = plsc.VectorSubcoreMesh(core_axis_name="c", subcore_axis_name="s")

    nch2 = 2 * ((nch + 1) // 2)

    @functools.partial(
        pl.kernel,
        out_type=jax.ShapeDtypeStruct((NC, NP, H), jnp.float32),
        mesh=mesh,
        cost_estimate=pl.CostEstimate(
            flops=3 * E * H, transcendentals=0,
            bytes_accessed=4 * E * H * 4 + 3 * E * 4 + NC * NP * H * 4),
        scratch_types=[
            [pltpu.VMEM((C,), jnp.int32)] * 2,
            [pltpu.VMEM((C,), jnp.int32)] * 2,
            [pltpu.VMEM((C,), jnp.float32)] * 2,
            [pltpu.VMEM((C, H), jnp.float32)] * 2,
            [pltpu.VMEM((C, H), jnp.float32)] * 2,
            pltpu.VMEM((ZR, H), jnp.float32),
            pltpu.VMEM_SHARED((NP, H), jnp.float32),
            [pltpu.SemaphoreType.DMA] * 2,
            [pltpu.SemaphoreType.DMA] * 2,
        ],
    )
    def k(p_hbm, q_hbm, ev_hbm, s_hbm, e_hbm, out_hbm, idx_s, idx_e, ebuf,
          buf_p, buf_q, zbuf, acc, isem, gsem):
        cid = lax.axis_index("c")
        sid = lax.axis_index("s")
        wid = sid * NC + cid
        base = wid * Ew
        r0 = sid * rows_per_tile

        def issue_idx(c, b):
            off = base + c * C
            pltpu.async_copy(s_hbm.at[pl.ds(off, C)], idx_s[b], isem[b])
            pltpu.async_copy(e_hbm.at[pl.ds(off, C)], idx_e[b], isem[b])
            pltpu.async_copy(ev_hbm.at[pl.ds(off, C)], ebuf[b], isem[b])

        def wait_idx(b):
            pltpu.make_async_copy(s_hbm.at[pl.ds(0, C)], idx_s[b], isem[b]).wait()
            pltpu.make_async_copy(e_hbm.at[pl.ds(0, C)], idx_e[b], isem[b]).wait()
            pltpu.make_async_copy(ev_hbm.at[pl.ds(0, C)], ebuf[b], isem[b]).wait()

        def issue_gather(b):
            pltpu.async_copy(p_hbm.at[idx_s[b]], buf_p[b], gsem[b])
            pltpu.async_copy(q_hbm.at[idx_e[b]], buf_q[b], gsem[b])

        def wait_gather(b):
            pltpu.make_async_copy(p_hbm.at[idx_s[b]], buf_p[b], gsem[b]).wait()
            pltpu.make_async_copy(q_hbm.at[idx_e[b]], buf_q[b], gsem[b]).wait()

        def zb(j, c2):
            for l in range(LG):
                zbuf[j, pl.ds(l * 16, 16)] = jnp.zeros((16,), jnp.float32)
            return c2

        lax.fori_loop(0, ZR, zb, 0)
        for i in range(nz):
            pltpu.sync_copy(zbuf, acc.at[pl.ds(r0 + i * ZR, ZR)])
        plsc.subcore_barrier()

        issue_idx(0, 0)
        issue_idx(1, 1)
        wait_idx(0)
        issue_gather(0)

        def step(kk, carry):
            for b in (0, 1):
                c = 2 * kk + b
                b1 = 1 - b

                @pl.when(c < nch)
                def _():
                    wait_gather(b)

                    @pl.when(c + 1 < nch)
                    def _():
                        wait_idx(b1)
                        issue_gather(b1)

                    def body(g, c2):
                        ev = ebuf[b][pl.ds(g * 16, 16)]
                        for l in range(16):
                            bv = jnp.full((16,), ev[l], jnp.float32)
                            j = g * 16 + l
                            for lg in range(LG):
                                sl = pl.ds(lg * 16, 16)
                                buf_p[b][j, sl] = buf_p[b][j, sl] * bv
                                buf_q[b][j, sl] = buf_q[b][j, sl] * bv
                        return c2

                    lax.fori_loop(0, C // 16, body, 0)
                    pltpu.sync_copy(buf_p[b], acc.at[idx_e[b]], add=True)
                    pltpu.sync_copy(buf_q[b], acc.at[idx_s[b]], add=True)

                    @pl.when(c + 2 < nch)
                    def _():
                        issue_idx(c + 2, b)
            return carry

        lax.fori_loop(0, nch2 // 2, step, 0)
        plsc.subcore_barrier()
        for i in range(nz):
            rr = r0 + i * ZR
            pltpu.sync_copy(acc.at[pl.ds(rr, ZR)], out_hbm.at[cid, pl.ds(rr, ZR)])

    return k(P, Q, e, start, end)[:, :N, :]


def kernel(x, edge_index, batch, params):
    N = x.shape[0]
    E = edge_index.shape[1]
    start = edge_index[0]
    end = edge_index[1]
    O = (batch[:, None] == jnp.arange(G, dtype=batch.dtype)[None, :]).astype(
        jnp.float32)

    RN = 1000
    NB = N // RN
    RE = 640
    E1 = (E // (2 * 2560) + 1) * 2560
    spans = [(0, E1), (E1, E - E1)]

    (wi, bi), = params["input"]
    (w1, b1), (w2, b2), (w3, b3), (w4, b4) = params["edge"]
    (wn1, bn1), (wn2, bn2), (wn3, bn3), (wn4, bn4) = params["node"]
    (wv1, bv1), (wv2, bv2), (wv3, bv3) = params["vp"]
    (wp1, bp1), (wp2, bp2), (wp3, bp3) = params["ip"]

    w1a, w1b, w1c = w1[:H], w1[H:2 * H], w1[2 * H:]
    wna, wnb, wnc, wnd = wn1[:H], wn1[H:2 * H], wn1[2 * H:3 * H], wn1[3 * H:]
    row = lambda v: v.reshape(1, -1)
    bi_, b1_, bn1_ = row(bi), row(b1), row(bn1)
    b2_, b3_, b4_ = row(b2), row(b3), row(b4)
    bn2_, bn3_, bn4_ = row(bn2), row(bn3), row(bn4)
    bv1_, bv2_, bv3_ = row(bv1), row(bv2), row(bv3)
    bp1_, bp2_, bp3_ = row(bp1), row(bp2), row(bp3)
    w4r = w4.reshape(1, H)

    wspec = _full((H, H))
    bspec = _full((1, H))
    gspec = _full((G, H))

    x1, ps, cnt = pl.pallas_call(
        _input_pool_kernel,
        grid=(NB,),
        in_specs=[_rows(RN, H), _rows(RN, G), wspec, bspec],
        out_specs=[_rows(RN, H), gspec, gspec],
        out_shape=[
            jax.ShapeDtypeStruct((N, H), jnp.float32),
            jax.ShapeDtypeStruct((G, H), jnp.float32),
            jax.ShapeDtypeStruct((G, H), jnp.float32),
        ],
    )(x, O, wi, bi_)

    w2b = w2.astype(jnp.bfloat16)
    w3b = w3.astype(jnp.bfloat16)
    vp_all = []
    s_h = [lax.slice_in_dim(start, o, o + l) for o, l in spans]
    e_h = [lax.slice_in_dim(end, o, o + l) for o, l in spans]

    xc = x1
    e_parts = None
    for _ in range(3):
        vp, cv, vd, A, B, P, Q, Rt = pl.pallas_call(
            _tables_kernel,
            grid=(NB,),
            in_specs=[_rows(RN, H), _rows(RN, G), gspec, gspec] +
                     [wspec, bspec] * 3 + [wspec, wspec] +
                     [wspec, wspec, bspec, wspec, wspec, wspec, bspec],
            out_specs=[gspec] * 3 + [_rows(RN, H)] * 5,
            out_shape=[jax.ShapeDtypeStruct((G, H), jnp.float32)] * 3 +
                      [jax.ShapeDtypeStruct((N, H), jnp.float32)] * 5,
        )(xc, O, ps, cnt, wv1, bv1_, wv2, bv2_, wv3, bv3_, w1c, wnd,
          w1a, w1b, b1_, wna, wnb, wnc, bn1_)
        vp_all.append(vp)

        h1s = [_sc_gather(A, B, s_h[i], e_h[i]) for i in range(2)]

        e_parts = [
            pl.pallas_call(
                _edge_tail_kernel,
                grid=(spans[i][1] // RE,),
                in_specs=[_rows(RE, H), _full((H, H)), bspec, _full((H, H)),
                          bspec, bspec, _full((1, 1))],
                out_specs=_rows(RE, 1),
                out_shape=jax.ShapeDtypeStruct((spans[i][1], 1), jnp.float32),
            )(h1s[i], w2b, b2_, w3b, b3_, w4r, b4_)
            for i in range(2)
        ]

        parts = [
            _sc_scatter(P, Q, e_parts[i].reshape(spans[i][1]), s_h[i], e_h[i])
            for i in range(2)
        ]

        xc, ps = pl.pallas_call(
            _node_kernel,
            grid=(NB,),
            in_specs=[_rows(RN, H)] * 6 + [_rows(RN, G)] +
                     [wspec, bspec] * 3,
            out_specs=[_rows(RN, H), gspec],
            out_shape=[
                jax.ShapeDtypeStruct((N, H), jnp.float32),
                jax.ShapeDtypeStruct((G, H), jnp.float32),
            ],
        )(parts[0][0], parts[0][1], parts[1][0], parts[1][1], Rt, xc, O,
          wn2, bn2_, wn3, bn3_, wn4, bn4_)

    ip = pl.pallas_call(
        _ip_kernel,
        grid=(1,),
        in_specs=[gspec] * 5 + [wspec, bspec] * 3 +
                 [_full((4 * H, H)), bspec] + [wspec, bspec] * 2,
        out_specs=gspec,
        out_shape=jax.ShapeDtypeStruct((G, H), jnp.float32),
    )(vp_all[0], vp_all[1], vp_all[2], ps, cnt, wv1, bv1_, wv2, bv2_, wv3,
      bv3_, wp1, bp1_, wp2, bp2_, wp3, bp3_)

    e_out = jnp.concatenate(
        [e_parts[0].reshape(spans[0][1]), e_parts[1].reshape(spans[1][1])])
    return (e_out, xc, ip)


# speedup vs baseline: 6.2411x; 1.1980x over previous
import functools

import jax
import jax.numpy as jnp
from jax import lax
from jax.experimental import pallas as pl
from jax.experimental.pallas import tpu as pltpu
from jax.experimental.pallas import tpu_sc as plsc

H = 128
G = 64
LG = H // 16


def _ln(x):
    n = x.shape[-1]
    m = jnp.sum(x, axis=-1, keepdims=True) / n
    s2 = jnp.sum(x * x, axis=-1, keepdims=True) / n
    v = s2 - m * m
    return (x - m) * lax.rsqrt(v + 1e-5)


def _dot(a, b):
    return jnp.dot(a, b, preferred_element_type=jnp.float32)


def _input_pool_kernel(x_ref, o_ref, wi_ref, bi_ref, x1_ref, ps_ref, cnt_ref):
    x1 = jnp.tanh(_ln(_dot(x_ref[...], wi_ref[...]) + bi_ref[...]))
    x1_ref[...] = x1
    ot = o_ref[...]

    @pl.when(pl.program_id(0) == 0)
    def _():
        ps_ref[...] = jnp.zeros_like(ps_ref)
        cnt_ref[...] = jnp.zeros_like(cnt_ref)

    ps_ref[...] += lax.dot_general(ot, x1, (((0,), (0,)), ((), ())),
                                   preferred_element_type=jnp.float32)
    cnt_ref[...] += lax.dot_general(
        ot, jnp.ones_like(x1), (((0,), (0,)), ((), ())),
        preferred_element_type=jnp.float32)


def _vp_mlp(ps, cnt, wv1, bv1, wv2, bv2, wv3, bv3):
    h = ps / cnt
    h = jnp.tanh(_ln(_dot(h, wv1[...]) + bv1[...]))
    h = jnp.tanh(_ln(_dot(h, wv2[...]) + bv2[...]))
    return jnp.tanh(_ln(_dot(h, wv3[...]) + bv3[...]))


def _tables_kernel(x_ref, o_ref, ps_ref, cnt_ref, wv1, bv1, wv2, bv2, wv3,
                   bv3, w1c, wnd, w1a, w1b, b1, wna, wnb, wnc, bn,
                   vp_ref, cv_ref, vd_ref, a_ref, b_ref, p_ref, q_ref, r_ref):
    @pl.when(pl.program_id(0) == 0)
    def _():
        vp = _vp_mlp(ps_ref[...], cnt_ref[...], wv1, bv1, wv2, bv2, wv3, bv3)
        vp_ref[...] = vp
        cv_ref[...] = _dot(vp, w1c[...])
        vd_ref[...] = _dot(vp, wnd[...])

    x = x_ref[...]
    o = o_ref[...]
    a_ref[...] = _dot(x, w1a[...]) + _dot(o, cv_ref[...]) + b1[...]
    b_ref[...] = _dot(x, w1b[...])
    p_ref[...] = _dot(x, wna[...])
    q_ref[...] = _dot(x, wnb[...])
    r_ref[...] = _dot(x, wnc[...]) + _dot(o, vd_ref[...]) + bn[...]


def _edge_tail_kernel(h1_ref, w2, b2, w3, b3, w4r, b4, e_ref):
    u = jnp.tanh(_ln(h1_ref[...].astype(jnp.float32)))
    u = _dot(u.astype(jnp.bfloat16), w2[...]) + b2[...]
    u = jnp.tanh(_ln(u))
    u = _dot(u.astype(jnp.bfloat16), w3[...]) + b3[...]
    u = jnp.tanh(_ln(u))
    logit = jnp.sum(u * w4r[...], axis=-1, keepdims=True) + b4[...]
    e_ref[...] = 1.0 / (1.0 + jnp.exp(-logit))


def _node_kernel(agga_ref, aggb_ref, aggc_ref, aggd_ref, rt_ref, xin_ref,
                 o_ref, wn2, bn2, wn3, bn3, wn4, bn4, xo_ref, ps_ref):
    h = jnp.tanh(_ln(agga_ref[...] + aggb_ref[...] + aggc_ref[...] +
                     aggd_ref[...] + rt_ref[...]))
    h = jnp.tanh(_ln(_dot(h, wn2[...]) + bn2[...]))
    h = jnp.tanh(_ln(_dot(h, wn3[...]) + bn3[...]))
    h = jnp.tanh(_ln(_dot(h, wn4[...]) + bn4[...]))

    @pl.when(pl.program_id(0) == 0)
    def _():
        ps_ref[...] = jnp.zeros_like(ps_ref)

    ps_ref[...] += lax.dot_general(o_ref[...], h, (((0,), (0,)), ((), ())),
                                   preferred_element_type=jnp.float32)
    xo_ref[...] = h + xin_ref[...]


def _ip_kernel(v0, v1, v2, ps_ref, cnt_ref, wv1, bv1, wv2, bv2, wv3, bv3,
               wp1, bp1, wp2, bp2, wp3, bp3, out_ref):
    v3 = _vp_mlp(ps_ref[...], cnt_ref[...], wv1, bv1, wv2, bv2, wv3, bv3)
    h = jnp.concatenate([v0[...], v1[...], v2[...], v3], axis=1)
    h = jnp.tanh(_ln(_dot(h, wp1[...]) + bp1[...]))
    h = jnp.tanh(_ln(_dot(h, wp2[...]) + bp2[...]))
    h = jnp.tanh(_ln(_dot(h, wp3[...]) + bp3[...]))
    out_ref[...] = h


def _full(shape):
    return pl.BlockSpec(shape, lambda i: (0,) * len(shape))


def _rows(bs, width):
    return pl.BlockSpec((bs, width), lambda i: (i, 0))


def _sc_gather(A, B, start, end):
    E = start.shape[0]
    info = plsc.get_sparse_core_info()
    NW = info.num_cores * info.num_subcores
    Ew = E // NW
    C = 128
    nch = 2 * ((Ew + 2 * C - 1) // (2 * C))
    mesh = plsc.VectorSubcoreMesh(core_axis_name="c", subcore_axis_name="s")

    @functools.partial(
        pl.kernel,
        out_type=jax.ShapeDtypeStruct((E, H), jnp.float32),
        mesh=mesh,
        cost_estimate=pl.CostEstimate(
            flops=E * H, transcendentals=0,
            bytes_accessed=3 * E * H * 4 + 2 * E * 4),
        scratch_types=[
            [pltpu.VMEM((C,), jnp.int32)] * 2,
            [pltpu.VMEM((C,), jnp.int32)] * 2,
            [pltpu.VMEM((C, H), jnp.float32)] * 2,
            [pltpu.VMEM((C, H), jnp.float32)] * 2,
            [pltpu.VMEM((C, H), jnp.float32)] * 2,
            [pltpu.SemaphoreType.DMA] * 2,
            [pltpu.SemaphoreType.DMA] * 2,
            [pltpu.SemaphoreType.DMA] * 2,
        ],
    )
    def k(a_hbm, b_hbm, s_hbm, e_hbm, out_hbm, idx_s, idx_e, buf_a, buf_b,
          buf_o, isem, gsem, osem):
        wid = lax.axis_index("s") * info.num_cores + lax.axis_index("c")
        base = wid * Ew
        off_of = lambda c: base + jnp.minimum(c * C, Ew - C)

        def issue_idx(c, b):
            pltpu.async_copy(s_hbm.at[pl.ds(off_of(c), C)], idx_s[b], isem[b])
            pltpu.async_copy(e_hbm.at[pl.ds(off_of(c), C)], idx_e[b], isem[b])

        def wait_idx(b):
            pltpu.make_async_copy(s_hbm.at[pl.ds(0, C)], idx_s[b], isem[b]).wait()
            pltpu.make_async_copy(e_hbm.at[pl.ds(0, C)], idx_e[b], isem[b]).wait()

        def issue_gather(b):
            pltpu.async_copy(a_hbm.at[idx_s[b]], buf_a[b], gsem[b])
            pltpu.async_copy(b_hbm.at[idx_e[b]], buf_b[b], gsem[b])

        def wait_gather(b):
            pltpu.make_async_copy(a_hbm.at[idx_s[b]], buf_a[b], gsem[b]).wait()
            pltpu.make_async_copy(b_hbm.at[idx_e[b]], buf_b[b], gsem[b]).wait()

        def wait_store(b):
            pltpu.make_async_copy(buf_o[b], out_hbm.at[pl.ds(0, C)], osem[b]).wait()

        issue_idx(0, 0)
        issue_idx(1, 1)
        wait_idx(0)
        issue_gather(0)

        def step(kk, carry):
            for b in (0, 1):
                c = 2 * kk + b
                b1 = 1 - b
                wait_gather(b)

                @pl.when(c + 1 < nch)
                def _():
                    wait_idx(b1)
                    issue_gather(b1)

                @pl.when(c + 2 < nch)
                def _():
                    issue_idx(c + 2, b)

                @pl.when(c >= 2)
                def _():
                    wait_store(b)

                def body(j, c2):
                    for l in range(LG):
                        sl = pl.ds(l * 16, 16)
                        buf_o[b][j, sl] = buf_a[b][j, sl] + buf_b[b][j, sl]
                    return c2

                lax.fori_loop(0, C, body, 0)
                pltpu.async_copy(buf_o[b], out_hbm.at[pl.ds(off_of(c), C)],
                                 osem[b])
            return carry

        lax.fori_loop(0, nch // 2, step, 0)
        wait_store(0)
        wait_store(1)

    return k(A, B, start, end)


def _sc_scatter(P, Q, e, start, end):
    E = start.shape[0]
    N = P.shape[0]
    info = plsc.get_sparse_core_info()
    NC, NS = info.num_cores, info.num_subcores
    NW = NC * NS
    Ew = E // NW
    C = 80
    nch = Ew // C
    ZR = 32
    rows_per_tile = (((N + NS - 1) // NS + ZR - 1) // ZR) * ZR
    NP = rows_per_tile * NS
    nz = rows_per_tile // ZR
    mesh = plsc.VectorSubcoreMesh(core_axis_name="c", subcore_axis_name="s")

    nch2 = 2 * ((nch + 1) // 2)

    @functools.partial(
        pl.kernel,
        out_type=jax.ShapeDtypeStruct((NC, NP, H), jnp.float32),
        mesh=mesh,
        cost_estimate=pl.CostEstimate(
            flops=3 * E * H, transcendentals=0,
            bytes_accessed=4 * E * H * 4 + 3 * E * 4 + NC * NP * H * 4),
        scratch_types=[
            [pltpu.VMEM((C,), jnp.int32)] * 2,
            [pltpu.VMEM((C,), jnp.int32)] * 2,
            [pltpu.VMEM((C,), jnp.float32)] * 2,
            [pltpu.VMEM((C, H), jnp.float32)] * 2,
            [pltpu.VMEM((C, H), jnp.float32)] * 2,
            pltpu.VMEM((ZR, H), jnp.float32),
            pltpu.VMEM_SHARED((NP, H), jnp.float32),
            [pltpu.SemaphoreType.DMA] * 2,
            [pltpu.SemaphoreType.DMA] * 2,
        ],
    )
    def k(p_hbm, q_hbm, ev_hbm, s_hbm, e_hbm, out_hbm, idx_s, idx_e, ebuf,
          buf_p, buf_q, zbuf, acc, isem, gsem):
        cid = lax.axis_index("c")
        sid = lax.axis_index("s")
        wid = sid * NC + cid
        base = wid * Ew
        r0 = sid * rows_per_tile

        def issue_idx(c, b):
            off = base + c * C
            pltpu.async_copy(s_hbm.at[pl.ds(off, C)], idx_s[b], isem[b])
            pltpu.async_copy(e_hbm.at[pl.ds(off, C)], idx_e[b], isem[b])
            pltpu.async_copy(ev_hbm.at[pl.ds(off, C)], ebuf[b], isem[b])

        def wait_idx(b):
            pltpu.make_async_copy(s_hbm.at[pl.ds(0, C)], idx_s[b], isem[b]).wait()
            pltpu.make_async_copy(e_hbm.at[pl.ds(0, C)], idx_e[b], isem[b]).wait()
            pltpu.make_async_copy(ev_hbm.at[pl.ds(0, C)], ebuf[b], isem[b]).wait()

        def issue_gather(b):
            pltpu.async_copy(p_hbm.at[idx_s[b]], buf_p[b], gsem[b])
            pltpu.async_copy(q_hbm.at[idx_e[b]], buf_q[b], gsem[b])

        def wait_gather(b):
            pltpu.make_async_copy(p_hbm.at[idx_s[b]], buf_p[b], gsem[b]).wait()
            pltpu.make_async_copy(q_hbm.at[idx_e[b]], buf_q[b], gsem[b]).wait()

        def zb(j, c2):
            for l in range(LG):
                zbuf[j, pl.ds(l * 16, 16)] = jnp.zeros((16,), jnp.float32)
            return c2

        lax.fori_loop(0, ZR, zb, 0)
        for i in range(nz):
            pltpu.sync_copy(zbuf, acc.at[pl.ds(r0 + i * ZR, ZR)])
        plsc.subcore_barrier()

        issue_idx(0, 0)
        issue_idx(1, 1)
        wait_idx(0)
        issue_gather(0)

        def step(kk, carry):
            for b in (0, 1):
                c = 2 * kk + b
                b1 = 1 - b

                @pl.when(c < nch)
                def _():
                    wait_gather(b)

                    @pl.when(c + 1 < nch)
                    def _():
                        wait_idx(b1)
                        issue_gather(b1)

                    def body(g, c2):
                        ev = ebuf[b][pl.ds(g * 16, 16)]
                        for l in range(16):
                            bv = jnp.full((16,), ev[l], jnp.float32)
                            j = g * 16 + l
                            for lg in range(LG):
                                sl = pl.ds(lg * 16, 16)
                                buf_p[b][j, sl] = buf_p[b][j, sl] * bv
                                buf_q[b][j, sl] = buf_q[b][j, sl] * bv
                        return c2

                    lax.fori_loop(0, C // 16, body, 0)
                    pltpu.sync_copy(buf_p[b], acc.at[idx_e[b]], add=True)
                    pltpu.sync_copy(buf_q[b], acc.at[idx_s[b]], add=True)

                    @pl.when(c + 2 < nch)
                    def _():
                        issue_idx(c + 2, b)
            return carry

        lax.fori_loop(0, nch2 // 2, step, 0)
        plsc.subcore_barrier()
        for i in range(nz):
            rr = r0 + i * ZR
            pltpu.sync_copy(acc.at[pl.ds(rr, ZR)], out_hbm.at[cid, pl.ds(rr, ZR)])

    return k(P, Q, e, start, end)[:, :N, :]


def kernel(x, edge_index, batch, params):
    N = x.shape[0]
    E = edge_index.shape[1]
    start = edge_index[0]
    end = edge_index[1]
    O = (batch[:, None] == jnp.arange(G, dtype=batch.dtype)[None, :]).astype(
        jnp.float32)

    RN = 1000
    NB = N // RN
    RE = 1280
    E1 = (E // (2 * 2560) + 1) * 2560
    spans = [(0, E1), (E1, E - E1)]

    (wi, bi), = params["input"]
    (w1, b1), (w2, b2), (w3, b3), (w4, b4) = params["edge"]
    (wn1, bn1), (wn2, bn2), (wn3, bn3), (wn4, bn4) = params["node"]
    (wv1, bv1), (wv2, bv2), (wv3, bv3) = params["vp"]
    (wp1, bp1), (wp2, bp2), (wp3, bp3) = params["ip"]

    w1a, w1b, w1c = w1[:H], w1[H:2 * H], w1[2 * H:]
    wna, wnb, wnc, wnd = wn1[:H], wn1[H:2 * H], wn1[2 * H:3 * H], wn1[3 * H:]
    row = lambda v: v.reshape(1, -1)
    bi_, b1_, bn1_ = row(bi), row(b1), row(bn1)
    b2_, b3_, b4_ = row(b2), row(b3), row(b4)
    bn2_, bn3_, bn4_ = row(bn2), row(bn3), row(bn4)
    bv1_, bv2_, bv3_ = row(bv1), row(bv2), row(bv3)
    bp1_, bp2_, bp3_ = row(bp1), row(bp2), row(bp3)
    w4r = w4.reshape(1, H)

    wspec = _full((H, H))
    bspec = _full((1, H))
    gspec = _full((G, H))

    x1, ps, cnt = pl.pallas_call(
        _input_pool_kernel,
        grid=(NB,),
        in_specs=[_rows(RN, H), _rows(RN, G), wspec, bspec],
        out_specs=[_rows(RN, H), gspec, gspec],
        out_shape=[
            jax.ShapeDtypeStruct((N, H), jnp.float32),
            jax.ShapeDtypeStruct((G, H), jnp.float32),
            jax.ShapeDtypeStruct((G, H), jnp.float32),
        ],
    )(x, O, wi, bi_)

    w2b = w2.astype(jnp.bfloat16)
    w3b = w3.astype(jnp.bfloat16)
    vp_all = []
    s_h = [lax.slice_in_dim(start, o, o + l) for o, l in spans]
    e_h = [lax.slice_in_dim(end, o, o + l) for o, l in spans]

    xc = x1
    e_parts = None
    for _ in range(3):
        vp, cv, vd, A, B, P, Q, Rt = pl.pallas_call(
            _tables_kernel,
            grid=(NB,),
            in_specs=[_rows(RN, H), _rows(RN, G), gspec, gspec] +
                     [wspec, bspec] * 3 + [wspec, wspec] +
                     [wspec, wspec, bspec, wspec, wspec, wspec, bspec],
            out_specs=[gspec] * 3 + [_rows(RN, H)] * 5,
            out_shape=[jax.ShapeDtypeStruct((G, H), jnp.float32)] * 3 +
                      [jax.ShapeDtypeStruct((N, H), jnp.float32)] * 5,
        )(xc, O, ps, cnt, wv1, bv1_, wv2, bv2_, wv3, bv3_, w1c, wnd,
          w1a, w1b, b1_, wna, wnb, wnc, bn1_)
        vp_all.append(vp)

        h1s = [_sc_gather(A, B, s_h[i], e_h[i]) for i in range(2)]

        e_parts = [
            pl.pallas_call(
                _edge_tail_kernel,
                grid=(spans[i][1] // RE,),
                in_specs=[_rows(RE, H), _full((H, H)), bspec, _full((H, H)),
                          bspec, bspec, _full((1, 1))],
                out_specs=_rows(RE, 1),
                out_shape=jax.ShapeDtypeStruct((spans[i][1], 1), jnp.float32),
            )(h1s[i], w2b, b2_, w3b, b3_, w4r, b4_)
            for i in range(2)
        ]

        parts = [
            _sc_scatter(P, Q, e_parts[i].reshape(spans[i][1]), s_h[i], e_h[i])
            for i in range(2)
        ]

        xc, ps = pl.pallas_call(
            _node_kernel,
            grid=(NB,),
            in_specs=[_rows(RN, H)] * 6 + [_rows(RN, G)] +
                     [wspec, bspec] * 3,
            out_specs=[_rows(RN, H), gspec],
            out_shape=[
                jax.ShapeDtypeStruct((N, H), jnp.float32),
                jax.ShapeDtypeStruct((G, H), jnp.float32),
            ],
        )(parts[0][0], parts[0][1], parts[1][0], parts[1][1], Rt, xc, O,
          wn2, bn2_, wn3, bn3_, wn4, bn4_)

    ip = pl.pallas_call(
        _ip_kernel,
        grid=(1,),
        in_specs=[gspec] * 5 + [wspec, bspec] * 3 +
                 [_full((4 * H, H)), bspec] + [wspec, bspec] * 2,
        out_specs=gspec,
        out_shape=jax.ShapeDtypeStruct((G, H), jnp.float32),
    )(vp_all[0], vp_all[1], vp_all[2], ps, cnt, wv1, bv1_, wv2, bv2_, wv3,
      bv3_, wp1, bp1_, wp2, bp2_, wp3, bp3_)

    e_out = jnp.concatenate(
        [e_parts[0].reshape(spans[0][1]), e_parts[1].reshape(spans[1][1])])
    return (e_out, xc, ip)


# speedup vs baseline: 6.7498x; 1.0815x over previous
import functools

import jax
import jax.numpy as jnp
from jax import lax
from jax.experimental import pallas as pl
from jax.experimental.pallas import tpu as pltpu
from jax.experimental.pallas import tpu_sc as plsc

H = 128
G = 64
LG = H // 16


def _ln(x):
    n = x.shape[-1]
    m = jnp.sum(x, axis=-1, keepdims=True) / n
    s2 = jnp.sum(x * x, axis=-1, keepdims=True) / n
    v = s2 - m * m
    return (x - m) * lax.rsqrt(v + 1e-5)


def _dot(a, b):
    return jnp.dot(a, b, preferred_element_type=jnp.float32)


def _input_pool_kernel(x_ref, o_ref, wi_ref, bi_ref, x1_ref, ps_ref, cnt_ref):
    x1 = jnp.tanh(_ln(_dot(x_ref[...], wi_ref[...]) + bi_ref[...]))
    x1_ref[...] = x1
    ot = o_ref[...]

    @pl.when(pl.program_id(0) == 0)
    def _():
        ps_ref[...] = jnp.zeros_like(ps_ref)
        cnt_ref[...] = jnp.zeros_like(cnt_ref)

    ps_ref[...] += lax.dot_general(ot, x1, (((0,), (0,)), ((), ())),
                                   preferred_element_type=jnp.float32)
    cnt_ref[...] += lax.dot_general(
        ot, jnp.ones_like(x1), (((0,), (0,)), ((), ())),
        preferred_element_type=jnp.float32)


def _vp_mlp(ps, cnt, wv1, bv1, wv2, bv2, wv3, bv3):
    h = ps / cnt
    h = jnp.tanh(_ln(_dot(h, wv1[...]) + bv1[...]))
    h = jnp.tanh(_ln(_dot(h, wv2[...]) + bv2[...]))
    return jnp.tanh(_ln(_dot(h, wv3[...]) + bv3[...]))


def _tables_kernel(x_ref, o_ref, ps_ref, cnt_ref, wv1, bv1, wv2, bv2, wv3,
                   bv3, w1c, wnd, w1a, w1b, b1, wna, wnb, wnc, bn,
                   vp_ref, cv_ref, vd_ref, a_ref, b_ref, p_ref, q_ref, r_ref):
    @pl.when(pl.program_id(0) == 0)
    def _():
        vp = _vp_mlp(ps_ref[...], cnt_ref[...], wv1, bv1, wv2, bv2, wv3, bv3)
        vp_ref[...] = vp
        cv_ref[...] = _dot(vp, w1c[...])
        vd_ref[...] = _dot(vp, wnd[...])

    x = x_ref[...]
    o = o_ref[...]
    a_ref[...] = _dot(x, w1a[...]) + _dot(o, cv_ref[...]) + b1[...]
    b_ref[...] = _dot(x, w1b[...])
    p_ref[...] = _dot(x, wna[...])
    q_ref[...] = _dot(x, wnb[...])
    r_ref[...] = _dot(x, wnc[...]) + _dot(o, vd_ref[...]) + bn[...]


def _edge_tail_kernel(h1_ref, w2, b2, w3, b3, w4r, b4, e_ref):
    u = jnp.tanh(_ln(h1_ref[...].astype(jnp.float32)))
    u = _dot(u.astype(jnp.bfloat16), w2[...]) + b2[...]
    u = jnp.tanh(_ln(u))
    u = _dot(u.astype(jnp.bfloat16), w3[...]) + b3[...]
    u = jnp.tanh(_ln(u))
    logit = jnp.sum(u * w4r[...], axis=-1, keepdims=True) + b4[...]
    e_ref[...] = 1.0 / (1.0 + jnp.exp(-logit))


def _node_kernel(agga_ref, aggb_ref, aggc_ref, aggd_ref, rt_ref, xin_ref,
                 o_ref, wn2, bn2, wn3, bn3, wn4, bn4, xo_ref, ps_ref):
    h = jnp.tanh(_ln(agga_ref[...] + aggb_ref[...] + aggc_ref[...] +
                     aggd_ref[...] + rt_ref[...]))
    h = jnp.tanh(_ln(_dot(h, wn2[...]) + bn2[...]))
    h = jnp.tanh(_ln(_dot(h, wn3[...]) + bn3[...]))
    h = jnp.tanh(_ln(_dot(h, wn4[...]) + bn4[...]))

    @pl.when(pl.program_id(0) == 0)
    def _():
        ps_ref[...] = jnp.zeros_like(ps_ref)

    ps_ref[...] += lax.dot_general(o_ref[...], h, (((0,), (0,)), ((), ())),
                                   preferred_element_type=jnp.float32)
    xo_ref[...] = h + xin_ref[...]


def _ip_kernel(v0, v1, v2, ps_ref, cnt_ref, wv1, bv1, wv2, bv2, wv3, bv3,
               wp1, bp1, wp2, bp2, wp3, bp3, out_ref):
    v3 = _vp_mlp(ps_ref[...], cnt_ref[...], wv1, bv1, wv2, bv2, wv3, bv3)
    h = jnp.concatenate([v0[...], v1[...], v2[...], v3], axis=1)
    h = jnp.tanh(_ln(_dot(h, wp1[...]) + bp1[...]))
    h = jnp.tanh(_ln(_dot(h, wp2[...]) + bp2[...]))
    h = jnp.tanh(_ln(_dot(h, wp3[...]) + bp3[...]))
    out_ref[...] = h


def _full(shape):
    return pl.BlockSpec(shape, lambda i: (0,) * len(shape))


def _rows(bs, width):
    return pl.BlockSpec((bs, width), lambda i: (i, 0))


def _sc_gather(A, B, start, end):
    E = start.shape[0]
    info = plsc.get_sparse_core_info()
    NW = info.num_cores * info.num_subcores
    Ew = E // NW
    C = 128
    nch = 2 * ((Ew + 2 * C - 1) // (2 * C))
    mesh = plsc.VectorSubcoreMesh(core_axis_name="c", subcore_axis_name="s")

    @functools.partial(
        pl.kernel,
        out_type=jax.ShapeDtypeStruct((E, H), jnp.float32),
        mesh=mesh,
        cost_estimate=pl.CostEstimate(
            flops=E * H, transcendentals=0,
            bytes_accessed=3 * E * H * 4 + 2 * E * 4),
        scratch_types=[
            [pltpu.VMEM((C,), jnp.int32)] * 2,
            [pltpu.VMEM((C,), jnp.int32)] * 2,
            [pltpu.VMEM((C, H), jnp.float32)] * 2,
            [pltpu.VMEM((C, H), jnp.float32)] * 2,
            [pltpu.VMEM((C, H), jnp.float32)] * 2,
            [pltpu.SemaphoreType.DMA] * 2,
            [pltpu.SemaphoreType.DMA] * 2,
            [pltpu.SemaphoreType.DMA] * 2,
        ],
    )
    def k(a_hbm, b_hbm, s_hbm, e_hbm, out_hbm, idx_s, idx_e, buf_a, buf_b,
          buf_o, isem, gsem, osem):
        wid = lax.axis_index("s") * info.num_cores + lax.axis_index("c")
        base = wid * Ew
        off_of = lambda c: base + jnp.minimum(c * C, Ew - C)

        def issue_idx(c, b):
            pltpu.async_copy(s_hbm.at[pl.ds(off_of(c), C)], idx_s[b], isem[b])
            pltpu.async_copy(e_hbm.at[pl.ds(off_of(c), C)], idx_e[b], isem[b])

        def wait_idx(b):
            pltpu.make_async_copy(s_hbm.at[pl.ds(0, C)], idx_s[b], isem[b]).wait()
            pltpu.make_async_copy(e_hbm.at[pl.ds(0, C)], idx_e[b], isem[b]).wait()

        def issue_gather(b):
            pltpu.async_copy(a_hbm.at[idx_s[b]], buf_a[b], gsem[b])
            pltpu.async_copy(b_hbm.at[idx_e[b]], buf_b[b], gsem[b])

        def wait_gather(b):
            pltpu.make_async_copy(a_hbm.at[idx_s[b]], buf_a[b], gsem[b]).wait()
            pltpu.make_async_copy(b_hbm.at[idx_e[b]], buf_b[b], gsem[b]).wait()

        def wait_store(b):
            pltpu.make_async_copy(buf_o[b], out_hbm.at[pl.ds(0, C)], osem[b]).wait()

        issue_idx(0, 0)
        issue_idx(1, 1)
        wait_idx(0)
        issue_gather(0)

        def step(kk, carry):
            for b in (0, 1):
                c = 2 * kk + b
                b1 = 1 - b
                wait_gather(b)

                @pl.when(c + 1 < nch)
                def _():
                    wait_idx(b1)
                    issue_gather(b1)

                @pl.when(c + 2 < nch)
                def _():
                    issue_idx(c + 2, b)

                @pl.when(c >= 2)
                def _():
                    wait_store(b)

                def body(j, c2):
                    for l in range(LG):
                        sl = pl.ds(l * 16, 16)
                        buf_o[b][j, sl] = buf_a[b][j, sl] + buf_b[b][j, sl]
                    return c2

                lax.fori_loop(0, C, body, 0)
                pltpu.async_copy(buf_o[b], out_hbm.at[pl.ds(off_of(c), C)],
                                 osem[b])
            return carry

        lax.fori_loop(0, nch // 2, step, 0)
        wait_store(0)
        wait_store(1)

    return k(A, B, start, end)


def _sc_scatter(P, Q, e, start, end):
    E = start.shape[0]
    N = P.shape[0]
    info = plsc.get_sparse_core_info()
    NC, NS = info.num_cores, info.num_subcores
    NW = NC * NS
    Ew = E // NW
    C = 80
    nch = Ew // C
    ZR = 32
    rows_per_tile = (((N + NS - 1) // NS + ZR - 1) // ZR) * ZR
    NP = rows_per_tile * NS
    nz = rows_per_tile // ZR
    mesh = plsc.VectorSubcoreMesh(core_axis_name="c", subcore_axis_name="s")

    nch2 = 2 * ((nch + 1) // 2)

    @functools.partial(
        pl.kernel,
        out_type=jax.ShapeDtypeStruct((NC, NP, H), jnp.float32),
        mesh=mesh,
        cost_estimate=pl.CostEstimate(
            flops=3 * E * H, transcendentals=0,
            bytes_accessed=4 * E * H * 4 + 3 * E * 4 + NC * NP * H * 4),
        scratch_types=[
            [pltpu.VMEM((C,), jnp.int32)] * 2,
            [pltpu.VMEM((C,), jnp.int32)] * 2,
            [pltpu.VMEM((C,), jnp.float32)] * 2,
            [pltpu.VMEM((C, H), jnp.float32)] * 2,
            [pltpu.VMEM((C, H), jnp.float32)] * 2,
            pltpu.VMEM((ZR, H), jnp.float32),
            pltpu.VMEM_SHARED((NP, H), jnp.float32),
            [pltpu.SemaphoreType.DMA] * 2,
            [pltpu.SemaphoreType.DMA] * 2,
        ],
    )
    def k(p_hbm, q_hbm, ev_hbm, s_hbm, e_hbm, out_hbm, idx_s, idx_e, ebuf,
          buf_p, buf_q, zbuf, acc, isem, gsem):
        cid = lax.axis_index("c")
        sid = lax.axis_index("s")
        wid = sid * NC + cid
        base = wid * Ew
        r0 = sid * rows_per_tile

        def issue_idx(c, b):
            off = base + c * C
            pltpu.async_copy(s_hbm.at[pl.ds(off, C)], idx_s[b], isem[b])
            pltpu.async_copy(e_hbm.at[pl.ds(off, C)], idx_e[b], isem[b])
            pltpu.async_copy(ev_hbm.at[pl.ds(off, C)], ebuf[b], isem[b])

        def wait_idx(b):
            pltpu.make_async_copy(s_hbm.at[pl.ds(0, C)], idx_s[b], isem[b]).wait()
            pltpu.make_async_copy(e_hbm.at[pl.ds(0, C)], idx_e[b], isem[b]).wait()
            pltpu.make_async_copy(ev_hbm.at[pl.ds(0, C)], ebuf[b], isem[b]).wait()

        def issue_gather(b):
            pltpu.async_copy(p_hbm.at[idx_s[b]], buf_p[b], gsem[b])
            pltpu.async_copy(q_hbm.at[idx_e[b]], buf_q[b], gsem[b])

        def wait_gather(b):
            pltpu.make_async_copy(p_hbm.at[idx_s[b]], buf_p[b], gsem[b]).wait()
            pltpu.make_async_copy(q_hbm.at[idx_e[b]], buf_q[b], gsem[b]).wait()

        def zb(j, c2):
            for l in range(LG):
                zbuf[j, pl.ds(l * 16, 16)] = jnp.zeros((16,), jnp.float32)
            return c2

        lax.fori_loop(0, ZR, zb, 0)
        for i in range(nz):
            pltpu.sync_copy(zbuf, acc.at[pl.ds(r0 + i * ZR, ZR)])
        plsc.subcore_barrier()

        issue_idx(0, 0)
        issue_idx(1, 1)
        wait_idx(0)
        issue_gather(0)

        def step(kk, carry):
            for b in (0, 1):
                c = 2 * kk + b
                b1 = 1 - b

                @pl.when(c < nch)
                def _():
                    wait_gather(b)

                    @pl.when(c + 1 < nch)
                    def _():
                        wait_idx(b1)
                        issue_gather(b1)

                    def body(g, c2):
                        ev = ebuf[b][pl.ds(g * 16, 16)]
                        for l in range(16):
                            bv = jnp.full((16,), ev[l], jnp.float32)
                            j = g * 16 + l
                            for lg in range(LG):
                                sl = pl.ds(lg * 16, 16)
                                buf_p[b][j, sl] = buf_p[b][j, sl] * bv
                                buf_q[b][j, sl] = buf_q[b][j, sl] * bv
                        return c2

                    lax.fori_loop(0, C // 16, body, 0)
                    pltpu.sync_copy(buf_p[b], acc.at[idx_e[b]], add=True)
                    pltpu.sync_copy(buf_q[b], acc.at[idx_s[b]], add=True)

                    @pl.when(c + 2 < nch)
                    def _():
                        issue_idx(c + 2, b)
            return carry

        lax.fori_loop(0, nch2 // 2, step, 0)
        plsc.subcore_barrier()
        for i in range(nz):
            rr = r0 + i * ZR
            pltpu.sync_copy(acc.at[pl.ds(rr, ZR)], out_hbm.at[cid, pl.ds(rr, ZR)])

    return k(P, Q, e, start, end)[:, :N, :]


def kernel(x, edge_index, batch, params):
    N = x.shape[0]
    E = edge_index.shape[1]
    start = edge_index[0]
    end = edge_index[1]
    O = (batch[:, None] == jnp.arange(G, dtype=batch.dtype)[None, :]).astype(
        jnp.float32)

    RN = 2000
    NB = N // RN
    RE = 2560
    E1 = (E // (2 * 2560) + 1) * 2560
    spans = [(0, E1), (E1, E - E1)]

    (wi, bi), = params["input"]
    (w1, b1), (w2, b2), (w3, b3), (w4, b4) = params["edge"]
    (wn1, bn1), (wn2, bn2), (wn3, bn3), (wn4, bn4) = params["node"]
    (wv1, bv1), (wv2, bv2), (wv3, bv3) = params["vp"]
    (wp1, bp1), (wp2, bp2), (wp3, bp3) = params["ip"]

    w1a, w1b, w1c = w1[:H], w1[H:2 * H], w1[2 * H:]
    wna, wnb, wnc, wnd = wn1[:H], wn1[H:2 * H], wn1[2 * H:3 * H], wn1[3 * H:]
    row = lambda v: v.reshape(1, -1)
    bi_, b1_, bn1_ = row(bi), row(b1), row(bn1)
    b2_, b3_, b4_ = row(b2), row(b3), row(b4)
    bn2_, bn3_, bn4_ = row(bn2), row(bn3), row(bn4)
    bv1_, bv2_, bv3_ = row(bv1), row(bv2), row(bv3)
    bp1_, bp2_, bp3_ = row(bp1), row(bp2), row(bp3)
    w4r = w4.reshape(1, H)

    wspec = _full((H, H))
    bspec = _full((1, H))
    gspec = _full((G, H))

    x1, ps, cnt = pl.pallas_call(
        _input_pool_kernel,
        grid=(NB,),
        in_specs=[_rows(RN, H), _rows(RN, G), wspec, bspec],
        out_specs=[_rows(RN, H), gspec, gspec],
        out_shape=[
            jax.ShapeDtypeStruct((N, H), jnp.float32),
            jax.ShapeDtypeStruct((G, H), jnp.float32),
            jax.ShapeDtypeStruct((G, H), jnp.float32),
        ],
    )(x, O, wi, bi_)

    w2b = w2.astype(jnp.bfloat16)
    w3b = w3.astype(jnp.bfloat16)
    vp_all = []
    s_h = [lax.slice_in_dim(start, o, o + l) for o, l in spans]
    e_h = [lax.slice_in_dim(end, o, o + l) for o, l in spans]

    xc = x1
    e_parts = None
    for _ in range(3):
        vp, cv, vd, A, B, P, Q, Rt = pl.pallas_call(
            _tables_kernel,
            grid=(NB,),
            in_specs=[_rows(RN, H), _rows(RN, G), gspec, gspec] +
                     [wspec, bspec] * 3 + [wspec, wspec] +
                     [wspec, wspec, bspec, wspec, wspec, wspec, bspec],
            out_specs=[gspec] * 3 + [_rows(RN, H)] * 5,
            out_shape=[jax.ShapeDtypeStruct((G, H), jnp.float32)] * 3 +
                      [jax.ShapeDtypeStruct((N, H), jnp.float32)] * 5,
        )(xc, O, ps, cnt, wv1, bv1_, wv2, bv2_, wv3, bv3_, w1c, wnd,
          w1a, w1b, b1_, wna, wnb, wnc, bn1_)
        vp_all.append(vp)

        h1s = [_sc_gather(A, B, s_h[i], e_h[i]) for i in range(2)]

        e_parts = [
            pl.pallas_call(
                _edge_tail_kernel,
                grid=(spans[i][1] // RE,),
                in_specs=[_rows(RE, H), _full((H, H)), bspec, _full((H, H)),
                          bspec, bspec, _full((1, 1))],
                out_specs=_rows(RE, 1),
                out_shape=jax.ShapeDtypeStruct((spans[i][1], 1), jnp.float32),
            )(h1s[i], w2b, b2_, w3b, b3_, w4r, b4_)
            for i in range(2)
        ]

        parts = [
            _sc_scatter(P, Q, e_parts[i].reshape(spans[i][1]), s_h[i], e_h[i])
            for i in range(2)
        ]

        xc, ps = pl.pallas_call(
            _node_kernel,
            grid=(NB,),
            in_specs=[_rows(RN, H)] * 6 + [_rows(RN, G)] +
                     [wspec, bspec] * 3,
            out_specs=[_rows(RN, H), gspec],
            out_shape=[
                jax.ShapeDtypeStruct((N, H), jnp.float32),
                jax.ShapeDtypeStruct((G, H), jnp.float32),
            ],
        )(parts[0][0], parts[0][1], parts[1][0], parts[1][1], Rt, xc, O,
          wn2, bn2_, wn3, bn3_, wn4, bn4_)

    ip = pl.pallas_call(
        _ip_kernel,
        grid=(1,),
        in_specs=[gspec] * 5 + [wspec, bspec] * 3 +
                 [_full((4 * H, H)), bspec] + [wspec, bspec] * 2,
        out_specs=gspec,
        out_shape=jax.ShapeDtypeStruct((G, H), jnp.float32),
    )(vp_all[0], vp_all[1], vp_all[2], ps, cnt, wv1, bv1_, wv2, bv2_, wv3,
      bv3_, wp1, bp1_, wp2, bp2_, wp3, bp3_)

    e_out = jnp.concatenate(
        [e_parts[0].reshape(spans[0][1]), e_parts[1].reshape(spans[1][1])])
    return (e_out, xc, ip)


# speedup vs baseline: 6.8711x; 1.0180x over previous
import functools

import jax
import jax.numpy as jnp
from jax import lax
from jax.experimental import pallas as pl
from jax.experimental.pallas import tpu as pltpu
from jax.experimental.pallas import tpu_sc as plsc

H = 128
G = 64
LG = H // 16


def _ln(x):
    n = x.shape[-1]
    m = jnp.sum(x, axis=-1, keepdims=True) / n
    s2 = jnp.sum(x * x, axis=-1, keepdims=True) / n
    v = s2 - m * m
    return (x - m) * lax.rsqrt(v + 1e-5)


def _dot(a, b):
    return jnp.dot(a, b, preferred_element_type=jnp.float32)


def _input_pool_kernel(x_ref, o_ref, wi_ref, bi_ref, x1_ref, ps_ref, cnt_ref):
    x1 = jnp.tanh(_ln(_dot(x_ref[...], wi_ref[...]) + bi_ref[...]))
    x1_ref[...] = x1
    ot = o_ref[...]

    @pl.when(pl.program_id(0) == 0)
    def _():
        ps_ref[...] = jnp.zeros_like(ps_ref)
        cnt_ref[...] = jnp.zeros_like(cnt_ref)

    ps_ref[...] += lax.dot_general(ot, x1, (((0,), (0,)), ((), ())),
                                   preferred_element_type=jnp.float32)
    cnt_ref[...] += lax.dot_general(
        ot, jnp.ones_like(x1), (((0,), (0,)), ((), ())),
        preferred_element_type=jnp.float32)


def _vp_mlp(ps, cnt, wv1, bv1, wv2, bv2, wv3, bv3):
    h = ps / cnt
    h = jnp.tanh(_ln(_dot(h, wv1[...]) + bv1[...]))
    h = jnp.tanh(_ln(_dot(h, wv2[...]) + bv2[...]))
    return jnp.tanh(_ln(_dot(h, wv3[...]) + bv3[...]))


def _tables_kernel(x_ref, o_ref, ps_ref, cnt_ref, wv1, bv1, wv2, bv2, wv3,
                   bv3, w1c, wnd, w1a, w1b, b1, wna, wnb, wnc, bn,
                   vp_ref, cv_ref, vd_ref, a_ref, b_ref, p_ref, q_ref, r_ref):
    @pl.when(pl.program_id(0) == 0)
    def _():
        vp = _vp_mlp(ps_ref[...], cnt_ref[...], wv1, bv1, wv2, bv2, wv3, bv3)
        vp_ref[...] = vp
        cv_ref[...] = _dot(vp, w1c[...])
        vd_ref[...] = _dot(vp, wnd[...])

    x = x_ref[...]
    o = o_ref[...]
    a_ref[...] = _dot(x, w1a[...]) + _dot(o, cv_ref[...]) + b1[...]
    b_ref[...] = _dot(x, w1b[...])
    p_ref[...] = _dot(x, wna[...])
    q_ref[...] = _dot(x, wnb[...])
    r_ref[...] = _dot(x, wnc[...]) + _dot(o, vd_ref[...]) + bn[...]


def _edge_tail_kernel(h1_ref, w2, b2, w3, b3, w4r, b4, e_ref):
    u = jnp.tanh(_ln(h1_ref[...].astype(jnp.float32)))
    u = _dot(u.astype(jnp.bfloat16), w2[...]) + b2[...]
    u = jnp.tanh(_ln(u))
    u = _dot(u.astype(jnp.bfloat16), w3[...]) + b3[...]
    u = jnp.tanh(_ln(u))
    logit = jnp.sum(u * w4r[...], axis=-1, keepdims=True) + b4[...]
    e_ref[...] = 1.0 / (1.0 + jnp.exp(-logit))


def _node_kernel(agga_ref, aggb_ref, aggc_ref, aggd_ref, rt_ref, xin_ref,
                 o_ref, wn2, bn2, wn3, bn3, wn4, bn4, xo_ref, ps_ref):
    h = jnp.tanh(_ln(agga_ref[...] + aggb_ref[...] + aggc_ref[...] +
                     aggd_ref[...] + rt_ref[...]))
    h = jnp.tanh(_ln(_dot(h, wn2[...]) + bn2[...]))
    h = jnp.tanh(_ln(_dot(h, wn3[...]) + bn3[...]))
    h = jnp.tanh(_ln(_dot(h, wn4[...]) + bn4[...]))

    @pl.when(pl.program_id(0) == 0)
    def _():
        ps_ref[...] = jnp.zeros_like(ps_ref)

    ps_ref[...] += lax.dot_general(o_ref[...], h, (((0,), (0,)), ((), ())),
                                   preferred_element_type=jnp.float32)
    xo_ref[...] = h + xin_ref[...]


def _ip_kernel(v0, v1, v2, ps_ref, cnt_ref, wv1, bv1, wv2, bv2, wv3, bv3,
               wp1, bp1, wp2, bp2, wp3, bp3, out_ref):
    v3 = _vp_mlp(ps_ref[...], cnt_ref[...], wv1, bv1, wv2, bv2, wv3, bv3)
    h = jnp.concatenate([v0[...], v1[...], v2[...], v3], axis=1)
    h = jnp.tanh(_ln(_dot(h, wp1[...]) + bp1[...]))
    h = jnp.tanh(_ln(_dot(h, wp2[...]) + bp2[...]))
    h = jnp.tanh(_ln(_dot(h, wp3[...]) + bp3[...]))
    out_ref[...] = h


def _full(shape):
    return pl.BlockSpec(shape, lambda i: (0,) * len(shape))


def _rows(bs, width):
    return pl.BlockSpec((bs, width), lambda i: (i, 0))


def _sc_gather(A, B, start, end):
    E = start.shape[0]
    info = plsc.get_sparse_core_info()
    NW = info.num_cores * info.num_subcores
    Ew = E // NW
    C = 128
    nch = 2 * ((Ew + 2 * C - 1) // (2 * C))
    mesh = plsc.VectorSubcoreMesh(core_axis_name="c", subcore_axis_name="s")

    @functools.partial(
        pl.kernel,
        out_type=jax.ShapeDtypeStruct((E, H), jnp.float32),
        mesh=mesh,
        cost_estimate=pl.CostEstimate(
            flops=E * H, transcendentals=0,
            bytes_accessed=3 * E * H * 4 + 2 * E * 4),
        scratch_types=[
            [pltpu.VMEM((C,), jnp.int32)] * 2,
            [pltpu.VMEM((C,), jnp.int32)] * 2,
            [pltpu.VMEM((C, H), jnp.float32)] * 2,
            [pltpu.VMEM((C, H), jnp.float32)] * 2,
            [pltpu.VMEM((C, H), jnp.float32)] * 2,
            [pltpu.SemaphoreType.DMA] * 2,
            [pltpu.SemaphoreType.DMA] * 2,
            [pltpu.SemaphoreType.DMA] * 2,
        ],
    )
    def k(a_hbm, b_hbm, s_hbm, e_hbm, out_hbm, idx_s, idx_e, buf_a, buf_b,
          buf_o, isem, gsem, osem):
        wid = lax.axis_index("s") * info.num_cores + lax.axis_index("c")
        base = wid * Ew
        off_of = lambda c: base + jnp.minimum(c * C, Ew - C)

        def issue_idx(c, b):
            pltpu.async_copy(s_hbm.at[pl.ds(off_of(c), C)], idx_s[b], isem[b])
            pltpu.async_copy(e_hbm.at[pl.ds(off_of(c), C)], idx_e[b], isem[b])

        def wait_idx(b):
            pltpu.make_async_copy(s_hbm.at[pl.ds(0, C)], idx_s[b], isem[b]).wait()
            pltpu.make_async_copy(e_hbm.at[pl.ds(0, C)], idx_e[b], isem[b]).wait()

        def issue_gather(b):
            pltpu.async_copy(a_hbm.at[idx_s[b]], buf_a[b], gsem[b])
            pltpu.async_copy(b_hbm.at[idx_e[b]], buf_b[b], gsem[b])

        def wait_gather(b):
            pltpu.make_async_copy(a_hbm.at[idx_s[b]], buf_a[b], gsem[b]).wait()
            pltpu.make_async_copy(b_hbm.at[idx_e[b]], buf_b[b], gsem[b]).wait()

        def wait_store(b):
            pltpu.make_async_copy(buf_o[b], out_hbm.at[pl.ds(0, C)], osem[b]).wait()

        issue_idx(0, 0)
        issue_idx(1, 1)
        wait_idx(0)
        issue_gather(0)

        def step(kk, carry):
            for b in (0, 1):
                c = 2 * kk + b
                b1 = 1 - b
                wait_gather(b)

                @pl.when(c + 1 < nch)
                def _():
                    wait_idx(b1)
                    issue_gather(b1)

                @pl.when(c + 2 < nch)
                def _():
                    issue_idx(c + 2, b)

                @pl.when(c >= 2)
                def _():
                    wait_store(b)

                def body(j, c2):
                    for l in range(LG):
                        sl = pl.ds(l * 16, 16)
                        buf_o[b][j, sl] = buf_a[b][j, sl] + buf_b[b][j, sl]
                    return c2

                lax.fori_loop(0, C, body, 0)
                pltpu.async_copy(buf_o[b], out_hbm.at[pl.ds(off_of(c), C)],
                                 osem[b])
            return carry

        lax.fori_loop(0, nch // 2, step, 0)
        wait_store(0)
        wait_store(1)

    return k(A, B, start, end)


def _sc_scatter(P, Q, e, start, end):
    E = start.shape[0]
    N = P.shape[0]
    info = plsc.get_sparse_core_info()
    NC, NS = info.num_cores, info.num_subcores
    NW = NC * NS
    Ew = E // NW
    C = 80
    nch = Ew // C
    ZR = 32
    rows_per_tile = (((N + NS - 1) // NS + ZR - 1) // ZR) * ZR
    NP = rows_per_tile * NS
    nz = rows_per_tile // ZR
    mesh = plsc.VectorSubcoreMesh(core_axis_name="c", subcore_axis_name="s")

    nch2 = 6 * ((nch + 5) // 6)

    @functools.partial(
        pl.kernel,
        out_type=jax.ShapeDtypeStruct((NC, NP, H), jnp.float32),
        mesh=mesh,
        cost_estimate=pl.CostEstimate(
            flops=3 * E * H, transcendentals=0,
            bytes_accessed=4 * E * H * 4 + 3 * E * 4 + NC * NP * H * 4),
        scratch_types=[
            [pltpu.VMEM((C,), jnp.int32)] * 3,
            [pltpu.VMEM((C,), jnp.int32)] * 3,
            [pltpu.VMEM((C,), jnp.float32)] * 3,
            [pltpu.VMEM((C, H), jnp.float32)] * 2,
            [pltpu.VMEM((C, H), jnp.float32)] * 2,
            pltpu.VMEM((ZR, H), jnp.float32),
            pltpu.VMEM_SHARED((NP, H), jnp.float32),
            [pltpu.SemaphoreType.DMA] * 3,
            [pltpu.SemaphoreType.DMA] * 2,
            [pltpu.SemaphoreType.DMA] * 2,
        ],
    )
    def k(p_hbm, q_hbm, ev_hbm, s_hbm, e_hbm, out_hbm, idx_s, idx_e, ebuf,
          buf_p, buf_q, zbuf, acc, isem, gsem, ssem):
        cid = lax.axis_index("c")
        sid = lax.axis_index("s")
        wid = sid * NC + cid
        base = wid * Ew
        r0 = sid * rows_per_tile

        def issue_idx(c, s3):
            off = base + c * C
            pltpu.async_copy(s_hbm.at[pl.ds(off, C)], idx_s[s3], isem[s3])
            pltpu.async_copy(e_hbm.at[pl.ds(off, C)], idx_e[s3], isem[s3])
            pltpu.async_copy(ev_hbm.at[pl.ds(off, C)], ebuf[s3], isem[s3])

        def wait_idx(s3):
            pltpu.make_async_copy(s_hbm.at[pl.ds(0, C)], idx_s[s3], isem[s3]).wait()
            pltpu.make_async_copy(e_hbm.at[pl.ds(0, C)], idx_e[s3], isem[s3]).wait()
            pltpu.make_async_copy(ev_hbm.at[pl.ds(0, C)], ebuf[s3], isem[s3]).wait()

        def issue_gather(b, s3):
            pltpu.async_copy(p_hbm.at[idx_s[s3]], buf_p[b], gsem[b])
            pltpu.async_copy(q_hbm.at[idx_e[s3]], buf_q[b], gsem[b])

        def wait_gather(b, s3):
            pltpu.make_async_copy(p_hbm.at[idx_s[s3]], buf_p[b], gsem[b]).wait()
            pltpu.make_async_copy(q_hbm.at[idx_e[s3]], buf_q[b], gsem[b]).wait()

        def issue_scat(b, s3):
            pltpu.async_copy(buf_p[b], acc.at[idx_e[s3]], ssem[b], add=True)
            pltpu.async_copy(buf_q[b], acc.at[idx_s[s3]], ssem[b], add=True)

        def wait_scat(b, s3):
            pltpu.make_async_copy(buf_p[b], acc.at[idx_e[s3]], ssem[b]).wait()
            pltpu.make_async_copy(buf_q[b], acc.at[idx_s[s3]], ssem[b]).wait()

        def zb(j, c2):
            for l in range(LG):
                zbuf[j, pl.ds(l * 16, 16)] = jnp.zeros((16,), jnp.float32)
            return c2

        lax.fori_loop(0, ZR, zb, 0)
        for i in range(nz):
            pltpu.sync_copy(zbuf, acc.at[pl.ds(r0 + i * ZR, ZR)])
        plsc.subcore_barrier()

        issue_idx(0, 0)
        issue_idx(1, 1)
        wait_idx(0)
        issue_gather(0, 0)

        def step(kk, carry):
            for u in range(6):
                c = 6 * kk + u
                b = u % 2
                b1 = 1 - b
                s3 = u % 3

                @pl.when(c < nch)
                def _():
                    wait_gather(b, s3)

                    @pl.when(c >= 1)
                    def _():
                        wait_scat(b1, (u - 1) % 3)

                    @pl.when(c + 1 < nch)
                    def _():
                        wait_idx((u + 1) % 3)
                        issue_gather(b1, (u + 1) % 3)

                    def body(g, c2):
                        ev = ebuf[s3][pl.ds(g * 16, 16)]
                        for l in range(16):
                            bv = jnp.full((16,), ev[l], jnp.float32)
                            j = g * 16 + l
                            for lg in range(LG):
                                sl = pl.ds(lg * 16, 16)
                                buf_p[b][j, sl] = buf_p[b][j, sl] * bv
                                buf_q[b][j, sl] = buf_q[b][j, sl] * bv
                        return c2

                    lax.fori_loop(0, C // 16, body, 0)
                    issue_scat(b, s3)

                    @pl.when(c + 2 < nch)
                    def _():
                        issue_idx(c + 2, (u + 2) % 3)
            return carry

        lax.fori_loop(0, nch2 // 6, step, 0)
        wait_scat((nch - 1) % 2, (nch - 1) % 3)
        plsc.subcore_barrier()
        for i in range(nz):
            rr = r0 + i * ZR
            pltpu.sync_copy(acc.at[pl.ds(rr, ZR)], out_hbm.at[cid, pl.ds(rr, ZR)])

    return k(P, Q, e, start, end)[:, :N, :]


def kernel(x, edge_index, batch, params):
    N = x.shape[0]
    E = edge_index.shape[1]
    start = edge_index[0]
    end = edge_index[1]
    O = (batch[:, None] == jnp.arange(G, dtype=batch.dtype)[None, :]).astype(
        jnp.float32)

    RN = 2000
    NB = N // RN
    RE = 2560
    E1 = (E // (2 * 2560) + 1) * 2560
    spans = [(0, E1), (E1, E - E1)]

    (wi, bi), = params["input"]
    (w1, b1), (w2, b2), (w3, b3), (w4, b4) = params["edge"]
    (wn1, bn1), (wn2, bn2), (wn3, bn3), (wn4, bn4) = params["node"]
    (wv1, bv1), (wv2, bv2), (wv3, bv3) = params["vp"]
    (wp1, bp1), (wp2, bp2), (wp3, bp3) = params["ip"]

    w1a, w1b, w1c = w1[:H], w1[H:2 * H], w1[2 * H:]
    wna, wnb, wnc, wnd = wn1[:H], wn1[H:2 * H], wn1[2 * H:3 * H], wn1[3 * H:]
    row = lambda v: v.reshape(1, -1)
    bi_, b1_, bn1_ = row(bi), row(b1), row(bn1)
    b2_, b3_, b4_ = row(b2), row(b3), row(b4)
    bn2_, bn3_, bn4_ = row(bn2), row(bn3), row(bn4)
    bv1_, bv2_, bv3_ = row(bv1), row(bv2), row(bv3)
    bp1_, bp2_, bp3_ = row(bp1), row(bp2), row(bp3)
    w4r = w4.reshape(1, H)

    wspec = _full((H, H))
    bspec = _full((1, H))
    gspec = _full((G, H))

    x1, ps, cnt = pl.pallas_call(
        _input_pool_kernel,
        grid=(NB,),
        in_specs=[_rows(RN, H), _rows(RN, G), wspec, bspec],
        out_specs=[_rows(RN, H), gspec, gspec],
        out_shape=[
            jax.ShapeDtypeStruct((N, H), jnp.float32),
            jax.ShapeDtypeStruct((G, H), jnp.float32),
            jax.ShapeDtypeStruct((G, H), jnp.float32),
        ],
    )(x, O, wi, bi_)

    w2b = w2.astype(jnp.bfloat16)
    w3b = w3.astype(jnp.bfloat16)
    vp_all = []
    s_h = [lax.slice_in_dim(start, o, o + l) for o, l in spans]
    e_h = [lax.slice_in_dim(end, o, o + l) for o, l in spans]

    xc = x1
    e_parts = None
    for _ in range(3):
        vp, cv, vd, A, B, P, Q, Rt = pl.pallas_call(
            _tables_kernel,
            grid=(NB,),
            in_specs=[_rows(RN, H), _rows(RN, G), gspec, gspec] +
                     [wspec, bspec] * 3 + [wspec, wspec] +
                     [wspec, wspec, bspec, wspec, wspec, wspec, bspec],
            out_specs=[gspec] * 3 + [_rows(RN, H)] * 5,
            out_shape=[jax.ShapeDtypeStruct((G, H), jnp.float32)] * 3 +
                      [jax.ShapeDtypeStruct((N, H), jnp.float32)] * 5,
        )(xc, O, ps, cnt, wv1, bv1_, wv2, bv2_, wv3, bv3_, w1c, wnd,
          w1a, w1b, b1_, wna, wnb, wnc, bn1_)
        vp_all.append(vp)

        h1s = [_sc_gather(A, B, s_h[i], e_h[i]) for i in range(2)]

        e_parts = [
            pl.pallas_call(
                _edge_tail_kernel,
                grid=(spans[i][1] // RE,),
                in_specs=[_rows(RE, H), _full((H, H)), bspec, _full((H, H)),
                          bspec, bspec, _full((1, 1))],
                out_specs=_rows(RE, 1),
                out_shape=jax.ShapeDtypeStruct((spans[i][1], 1), jnp.float32),
            )(h1s[i], w2b, b2_, w3b, b3_, w4r, b4_)
            for i in range(2)
        ]

        parts = [
            _sc_scatter(P, Q, e_parts[i].reshape(spans[i][1]), s_h[i], e_h[i])
            for i in range(2)
        ]

        xc, ps = pl.pallas_call(
            _node_kernel,
            grid=(NB,),
            in_specs=[_rows(RN, H)] * 6 + [_rows(RN, G)] +
                     [wspec, bspec] * 3,
            out_specs=[_rows(RN, H), gspec],
            out_shape=[
                jax.ShapeDtypeStruct((N, H), jnp.float32),
                jax.ShapeDtypeStruct((G, H), jnp.float32),
            ],
        )(parts[0][0], parts[0][1], parts[1][0], parts[1][1], Rt, xc, O,
          wn2, bn2_, wn3, bn3_, wn4, bn4_)

    ip = pl.pallas_call(
        _ip_kernel,
        grid=(1,),
        in_specs=[gspec] * 5 + [wspec, bspec] * 3 +
                 [_full((4 * H, H)), bspec] + [wspec, bspec] * 2,
        out_specs=gspec,
        out_shape=jax.ShapeDtypeStruct((G, H), jnp.float32),
    )(vp_all[0], vp_all[1], vp_all[2], ps, cnt, wv1, bv1_, wv2, bv2_, wv3,
      bv3_, wp1, bp1_, wp2, bp2_, wp3, bp3_)

    e_out = jnp.concatenate(
        [e_parts[0].reshape(spans[0][1]), e_parts[1].reshape(spans[1][1])])
    return (e_out, xc, ip)
